# Initial kernel scaffold; baseline (speedup 1.0000x reference)
import jax
import jax.numpy as jnp
from jax.experimental import pallas as pl


def kernel(x, edge_index, edge_attr, batch, params):
    raise NotImplementedError("write your pallas kernel here")


# speedup vs baseline: 8.7730x; 8.7730x over previous
import functools

import jax
import jax.numpy as jnp
from jax import lax
from jax.experimental import pallas as pl
from jax.experimental.pallas import tpu as pltpu

NN = 10000
NE = 320000
DIN = 128
DE = 4
NH = 4
DH = 64
HC = 256
NG = 8
_BN_K = 1.0 / (1.0 + 1e-5) ** 0.5

_INTERP = False

NB = 1000
EB = 4000


def _dot(a, b):
    return jnp.dot(a, b, preferred_element_type=jnp.float32)


def _full(shape):
    return pl.BlockSpec(shape, lambda *a: tuple(0 for _ in shape))


def _rows(shape):
    return pl.BlockSpec(shape, lambda i: (i,) + tuple(0 for _ in shape[1:]))


def _pre_body(x_r, win_r, bin_r, wl_r, bl_r, wr_r, br_r, xl_r, xr_r):
    h = _dot(x_r[...], win_r[...]) + bin_r[...]
    xl_r[...] = _dot(h, wl_r[...]) + bl_r[...]
    xr_r[...] = _dot(h, wr_r[...]) + br_r[...]


def _pre(x, win, bin_, wl, bl, wr, br):
    return pl.pallas_call(
        _pre_body,
        grid=(NN // NB,),
        in_specs=[_rows((NB, DIN)), _full((DIN, DH)), _full((1, DH)),
                  _full((DH, HC)), _full((1, HC)), _full((DH, HC)), _full((1, HC))],
        out_specs=[_rows((NB, HC)), _rows((NB, HC))],
        out_shape=[jax.ShapeDtypeStruct((NN, HC), jnp.float32)] * 2,
        interpret=_INTERP,
    )(x, win, bin_, wl, bl, wr, br)


def _alpha_body(q_r, ea_r, we_r, att_r, s_r):
    m = q_r[...] + _dot(ea_r[...], we_r[...])
    m = jnp.where(m >= 0.0, m, 0.2 * m)
    ma = m * att_r[...]
    parts = [jnp.sum(ma[:, h * DH:(h + 1) * DH], axis=1, keepdims=True)
             for h in range(NH)]
    s_r[...] = jnp.exp(jnp.concatenate(parts, axis=1))


def _alpha(q, ea, we, att_flat):
    return pl.pallas_call(
        _alpha_body,
        grid=(NE // EB,),
        in_specs=[_rows((EB, HC)), _rows((EB, DE)), _full((DE, HC)), _full((1, HC))],
        out_specs=_rows((EB, NH)),
        out_shape=jax.ShapeDtypeStruct((NE, NH), jnp.float32),
        interpret=_INTERP,
    )(q, ea, we, att_flat)


def _norm(acc, den256, gb, lng, lnb):
    o = acc / (den256 + 1e-16) + gb
    mu = jnp.mean(o, axis=1, keepdims=True)
    v = jnp.mean((o - mu) ** 2, axis=1, keepdims=True)
    o = (o - mu) * lax.rsqrt(v + 1e-5) * lng + lnb
    return jnp.where(o > 0.0, o, jnp.exp(o) - 1.0)


def _mid_body(acc_r, den_r, hsel_r, gb_r, lng_r, lnb_r, wl_r, bl_r, wr_r, br_r,
              h0_r, xl_r, xr_r):
    den256 = _dot(den_r[...], hsel_r[...])
    h0 = _norm(acc_r[...], den256, gb_r[...], lng_r[...], lnb_r[...])
    h0_r[...] = h0
    xl_r[...] = _dot(h0, wl_r[...]) + bl_r[...]
    xr_r[...] = _dot(h0, wr_r[...]) + br_r[...]


def _mid(acc, den, hsel, gb, lng, lnb, wl, bl, wr, br):
    return pl.pallas_call(
        _mid_body,
        grid=(NN // NB,),
        in_specs=[_rows((NB, HC)), _rows((NB, NH)), _full((NH, HC)),
                  _full((1, HC)), _full((1, HC)), _full((1, HC)),
                  _full((HC, HC)), _full((1, HC)), _full((HC, HC)), _full((1, HC))],
        out_specs=[_rows((NB, HC))] * 3,
        out_shape=[jax.ShapeDtypeStruct((NN, HC), jnp.float32)] * 3,
        interpret=_INTERP,
    )(acc, den, hsel, gb, lng, lnb, wl, bl, wr, br)


def _post1_body(acc_r, den_r, hsel_r, gb_r, lng_r, lnb_r, res_r, h1_r):
    den256 = _dot(den_r[...], hsel_r[...])
    h1_r[...] = _norm(acc_r[...], den256, gb_r[...], lng_r[...], lnb_r[...]) + res_r[...]


def _post1(acc, den, hsel, gb, lng, lnb, res):
    return pl.pallas_call(
        _post1_body,
        grid=(NN // NB,),
        in_specs=[_rows((NB, HC)), _rows((NB, NH)), _full((NH, HC)),
                  _full((1, HC)), _full((1, HC)), _full((1, HC)), _rows((NB, HC))],
        out_specs=_rows((NB, HC)),
        out_shape=jax.ShapeDtypeStruct((NN, HC), jnp.float32),
        interpret=_INTERP,
    )(acc, den, hsel, gb, lng, lnb, res)


def _c1_body(hp_r, h_r, hn_r, pm_r, nm_r, w0_r, w1_r, w2_r, cb_r, g_r, b_r, t1_r):
    y = (_dot(hp_r[...] * pm_r[...], w0_r[...]) + _dot(h_r[...], w1_r[...])
         + _dot(hn_r[...] * nm_r[...], w2_r[...]) + cb_r[...])
    t1_r[...] = jnp.maximum(y * _BN_K * g_r[...] + b_r[...], 0.0)


def _conv1(hp, h, hn, pm, nm, w0, w1, w2, cb, g, b):
    return pl.pallas_call(
        _c1_body,
        grid=(NN // NB,),
        in_specs=[_rows((NB, HC))] * 3 + [_rows((NB, 1))] * 2
                 + [_full((HC, HC))] * 3 + [_full((1, HC))] * 3,
        out_specs=_rows((NB, HC)),
        out_shape=jax.ShapeDtypeStruct((NN, HC), jnp.float32),
        interpret=_INTERP,
    )(hp, h, hn, pm, nm, w0, w1, w2, cb, g, b)


def _c2_body(tp_r, t_r, tn_r, pm_r, nm_r, sel_r, lw2_r, res_r,
             w0_r, w1_r, w2_r, cb_r, g_r, b_r, t2_r):
    y = (_dot(tp_r[...] * pm_r[...], w0_r[...]) + _dot(t_r[...], w1_r[...])
         + _dot(tn_r[...] * nm_r[...], w2_r[...])
         + _dot(sel_r[...], lw2_r[...]) + cb_r[...])
    t2_r[...] = jnp.maximum(y * _BN_K * g_r[...] + b_r[...] + res_r[...], 0.0)


def _conv2(tp, t, tn, pm, nm, sel, lw2, res, w0, w1, w2, cb, g, b):
    return pl.pallas_call(
        _c2_body,
        grid=(NN // NB,),
        in_specs=[_rows((NB, HC))] * 3 + [_rows((NB, 1))] * 2
                 + [_rows((NB, NG)), _full((NG, HC)), _rows((NB, HC))]
                 + [_full((HC, HC))] * 3 + [_full((1, HC))] * 3,
        out_specs=_rows((NB, HC)),
        out_shape=jax.ShapeDtypeStruct((NN, HC), jnp.float32),
        interpret=_INTERP,
    )(tp, t, tn, pm, nm, sel, lw2, res, w0, w1, w2, cb, g, b)


def _leak_body(hl_r, t1l_r, cnt_r, tv_r, w0c1_r, c1b_r, g1_r, b1_r,
               w0c2_r, w1c2_r, w2c2_r, c2b_r, g2_r, b2_r,
               lw2_r, psum_r, pmax_r):
    cb = cnt_r[...]
    tv = tv_r[...]
    l1 = jnp.maximum((_dot(hl_r[...], w0c1_r[...]) + c1b_r[...]) * _BN_K
                     * g1_r[...] + b1_r[...], 0.0)
    m1 = ((cb > 0.0) & (cb < float(NN)) & (cb < tv)).astype(jnp.float32)
    l1m = l1 * m1
    p1 = jnp.maximum((_dot(t1l_r[...], w0c2_r[...]) + _dot(l1m, w1c2_r[...])
                      + c2b_r[...]) * _BN_K * g2_r[...] + b2_r[...], 0.0) * m1
    m2 = ((cb + 1.0 < tv) & (cb + 1.0 < float(NN))).astype(jnp.float32)
    p2 = jnp.maximum((_dot(l1m, w0c2_r[...]) + c2b_r[...]) * _BN_K
                     * g2_r[...] + b2_r[...], 0.0) * m2
    lw2_r[...] = _dot(l1m, w2c2_r[...])
    psum_r[...] = p1 + p2
    pmax_r[...] = jnp.maximum(p1, p2)


def _leak(hl, t1l, cnt, tv, w0c1, c1b, g1, b1, w0c2, w1c2, w2c2, c2b, g2, b2):
    return pl.pallas_call(
        _leak_body,
        in_specs=[_full((NG, HC))] * 2 + [_full((NG, 1)), _full((1, 1)),
                  _full((HC, HC)), _full((1, HC)), _full((1, HC)), _full((1, HC)),
                  _full((HC, HC)), _full((HC, HC)), _full((HC, HC)),
                  _full((1, HC)), _full((1, HC)), _full((1, HC))],
        out_specs=[_full((NG, HC))] * 3,
        out_shape=[jax.ShapeDtypeStruct((NG, HC), jnp.float32)] * 3,
        interpret=_INTERP,
    )(hl, t1l, cnt, tv, w0c1, c1b, g1, b1, w0c2, w1c2, w2c2, c2b, g2, b2)


def _pool_body(t2_r, oht_r, oh_r, psum_r, pmax_r, tv_r,
               w1_r, b1_r, w2_r, b2_r, w3_r, b3_r, out_r):
    t2 = t2_r[...]
    sums = _dot(oht_r[...], t2)
    oh = oh_r[...]
    maxs = [jnp.max(t2 * oh[:, b:b + 1], axis=0, keepdims=True)
            for b in range(NG)]
    mx = jnp.concatenate(maxs, axis=0)
    mean = (sums + psum_r[...]) / tv_r[...]
    mx = jnp.maximum(mx, pmax_r[...])
    g = jnp.concatenate([mean, mx], axis=1)
    g = jnp.maximum(_dot(g, w1_r[...]) + b1_r[...], 0.0)
    g = jnp.maximum(_dot(g, w2_r[...]) + b2_r[...], 0.0)
    out_r[...] = _dot(g, w3_r[...]) + b3_r[...]


def _pool(t2, oht, oh, psum, pmax, tv, w1, b1, w2, b2, w3, b3):
    return pl.pallas_call(
        _pool_body,
        in_specs=[_full((NN, HC)), _full((NG, NN)), _full((NN, NG)),
                  _full((NG, HC)), _full((NG, HC)), _full((1, 1)),
                  _full((2 * HC, HC)), _full((1, HC)),
                  _full((HC, DH)), _full((1, DH)),
                  _full((DH, 1)), _full((1, 1))],
        out_specs=_full((NG, 1)),
        out_shape=jax.ShapeDtypeStruct((NG, 1), jnp.float32),
        interpret=_INTERP,
    )(t2, oht, oh, psum, pmax, tv, w1, b1, w2, b2, w3, b3)


def _edge_aggregate(xl, xr, ea, src, dst, we, att_flat):
    q = xl[src] + xr[dst]
    s = _alpha(q, ea, we, att_flat)
    xle = xl[src]
    contrib = (xle.reshape(NE, NH, DH) * s[:, :, None]).reshape(NE, HC)
    acc = jax.ops.segment_sum(contrib, dst, num_segments=NN)
    den = jax.ops.segment_sum(s, dst, num_segments=NN)
    return acc, den


def kernel(x, edge_index, edge_attr, batch, params):
    p = params
    src = edge_index[0]
    dst = edge_index[1]
    r1 = lambda a: a.reshape(1, -1)

    hsel = (jnp.arange(HC, dtype=jnp.int32)[None, :] // DH
            == jnp.arange(NH, dtype=jnp.int32)[:, None]).astype(jnp.float32)

    xl0, xr0 = _pre(x, p['Win'], r1(p['bin']), p['g0_Wl'], r1(p['g0_bl']),
                    p['g0_Wr'], r1(p['g0_br']))
    acc0, den0 = _edge_aggregate(xl0, xr0, edge_attr, src, dst,
                                 p['g0_We'], r1(p['g0_att']))
    h0, xl1, xr1 = _mid(acc0, den0, hsel, r1(p['g0_bias']), r1(p['ln0_g']),
                        r1(p['ln0_b']), p['g1_Wl'], r1(p['g1_bl']),
                        p['g1_Wr'], r1(p['g1_br']))
    acc1, den1 = _edge_aggregate(xl1, xr1, edge_attr, src, dst,
                                 p['g1_We'], r1(p['g1_att']))
    h1 = _post1(acc1, den1, hsel, r1(p['g1_bias']), r1(p['ln1_g']),
                r1(p['ln1_b']), h0)

    idx = jnp.arange(NN, dtype=jnp.int32)
    o = jnp.searchsorted(batch, jnp.arange(NG + 1, dtype=jnp.int32)).astype(jnp.int32)
    cnt = (o[1:] - o[:-1]).astype(jnp.float32)
    tv = jnp.max(cnt)
    same_next = (batch[1:] == batch[:-1])
    pm = jnp.concatenate([jnp.zeros((1,), jnp.bool_), same_next]).astype(jnp.float32)[:, None]
    nm = jnp.concatenate([same_next, jnp.zeros((1,), jnp.bool_)]).astype(jnp.float32)[:, None]
    z1 = jnp.zeros((1, HC), jnp.float32)
    shift = lambda a: (jnp.concatenate([z1, a[:-1]], 0), jnp.concatenate([a[1:], z1], 0))

    lastrow = jnp.clip(o[1:] - 1, 0, NN - 1)
    sel = (idx[:, None] == (o[1:] - 1)[None, :]).astype(jnp.float32)
    oh = (batch[:, None] == jnp.arange(NG, dtype=jnp.int32)[None, :]).astype(jnp.float32)
    oht = oh.T

    wc1 = [p['c1_w'][:, :, k].T for k in range(3)]
    wc2 = [p['c2_w'][:, :, k].T for k in range(3)]

    h1p, h1n = shift(h1)
    t1 = _conv1(h1p, h1, h1n, pm, nm, wc1[0], wc1[1], wc1[2],
                r1(p['c1_b']), r1(p['bn1_g']), r1(p['bn1_b']))

    hl = h1[lastrow]
    t1l = t1[lastrow]
    lw2, psum, pmax = _leak(hl, t1l, cnt[:, None], tv.reshape(1, 1),
                            wc1[0], r1(p['c1_b']), r1(p['bn1_g']), r1(p['bn1_b']),
                            wc2[0], wc2[1], wc2[2],
                            r1(p['c2_b']), r1(p['bn2_g']), r1(p['bn2_b']))

    t1p, t1n = shift(t1)
    t2 = _conv2(t1p, t1, t1n, pm, nm, sel, lw2, h1,
                wc2[0], wc2[1], wc2[2],
                r1(p['c2_b']), r1(p['bn2_g']), r1(p['bn2_b']))

    out = _pool(t2, oht, oh, psum, pmax, tv.reshape(1, 1),
                p['w1'], r1(p['b1']), p['w2'], r1(p['b2']),
                p['w3'], r1(p['b3']))
    return out.reshape(NG)


# speedup vs baseline: 16.2013x; 1.8467x over previous
import functools

import jax
import jax.numpy as jnp
from jax import lax
from jax.experimental import pallas as pl
from jax.experimental.pallas import tpu as pltpu
from jax.experimental.pallas import tpu_sc as plsc

NN = 10000
NE = 320000
DIN = 128
DE = 4
NH = 4
DH = 64
HC = 256
NG = 8
_BN_K = 1.0 / (1.0 + 1e-5) ** 0.5

_INTERP = False

NB = 1000
EB = 4000


def _dot(a, b):
    return jnp.dot(a, b, preferred_element_type=jnp.float32)


def _dot_hi(a, b):
    return jnp.dot(a, b, preferred_element_type=jnp.float32,
                   precision=lax.Precision.HIGHEST)


def _full(shape):
    return pl.BlockSpec(shape, lambda *a: tuple(0 for _ in shape))


def _rows(shape):
    return pl.BlockSpec(shape, lambda i: (i,) + tuple(0 for _ in shape[1:]))


def _pre_body(x_r, win_r, bin_r, wl_r, bl_r, wr_r, br_r, xl_r, xr_r):
    h = _dot(x_r[...], win_r[...]) + bin_r[...]
    xl_r[...] = _dot(h, wl_r[...]) + bl_r[...]
    xr_r[...] = _dot(h, wr_r[...]) + br_r[...]


def _pre(x, win, bin_, wl, bl, wr, br):
    return pl.pallas_call(
        _pre_body,
        grid=(NN // NB,),
        in_specs=[_rows((NB, DIN)), _full((DIN, DH)), _full((1, DH)),
                  _full((DH, HC)), _full((1, HC)), _full((DH, HC)), _full((1, HC))],
        out_specs=[_rows((NB, HC)), _rows((NB, HC))],
        out_shape=[jax.ShapeDtypeStruct((NN, HC), jnp.float32)] * 2,
        interpret=_INTERP,
    )(x, win, bin_, wl, bl, wr, br)


def _alpha_body(q_r, ea_r, we_r, att_r, s_r):
    m = q_r[...] + _dot(ea_r[...], we_r[...])
    m = jnp.where(m >= 0.0, m, 0.2 * m)
    ma = m * att_r[...]
    parts = [jnp.sum(ma[:, h * DH:(h + 1) * DH], axis=1, keepdims=True)
             for h in range(NH)]
    s_r[...] = jnp.exp(jnp.concatenate(parts, axis=1))


def _alpha(q, ea, we, att_flat):
    return pl.pallas_call(
        _alpha_body,
        grid=(NE // EB,),
        in_specs=[_rows((EB, HC)), _rows((EB, DE)), _full((DE, HC)), _full((1, HC))],
        out_specs=_rows((EB, NH)),
        out_shape=jax.ShapeDtypeStruct((NE, NH), jnp.float32),
        interpret=_INTERP,
    )(q, ea, we, att_flat)


def _norm(acc, den256, gb, lng, lnb):
    o = acc / (den256 + 1e-16) + gb
    mu = jnp.mean(o, axis=1, keepdims=True)
    v = jnp.mean((o - mu) ** 2, axis=1, keepdims=True)
    o = (o - mu) / jnp.sqrt(v + 1e-5) * lng + lnb
    return jnp.where(o > 0.0, o, jnp.exp(o) - 1.0)


def _agg_norm(lo, hi, den16, gb, lng, lnb):
    acc = jnp.concatenate([lo, hi], axis=1)
    o = jnp.concatenate(
        [acc[:, h * DH:(h + 1) * DH] / (den16[:, h:h + 1] + 1e-16)
         for h in range(NH)], axis=1) + gb
    mu = jnp.mean(o, axis=1, keepdims=True)
    v = jnp.mean((o - mu) ** 2, axis=1, keepdims=True)
    o = (o - mu) / jnp.sqrt(v + 1e-5) * lng + lnb
    return jnp.where(o > 0.0, o, jnp.exp(o) - 1.0)


def _mid_body(lo_r, hi_r, den_r, gb_r, lng_r, lnb_r, wl_r, bl_r, wr_r, br_r,
              h0_r, xl_r, xr_r):
    h0 = _agg_norm(lo_r[...], hi_r[...], den_r[...], gb_r[...],
                   lng_r[...], lnb_r[...])
    h0_r[...] = h0
    xl_r[...] = _dot(h0, wl_r[...]) + bl_r[...]
    xr_r[...] = _dot(h0, wr_r[...]) + br_r[...]


def _mid(lo, hi, den, gb, lng, lnb, wl, bl, wr, br):
    return pl.pallas_call(
        _mid_body,
        grid=(NN // NB,),
        in_specs=[_rows((NB, HW)), _rows((NB, HW)), _rows((NB, 16)),
                  _full((1, HC)), _full((1, HC)), _full((1, HC)),
                  _full((HC, HC)), _full((1, HC)), _full((HC, HC)), _full((1, HC))],
        out_specs=[_rows((NB, HC))] * 3,
        out_shape=[jax.ShapeDtypeStruct((NN, HC), jnp.float32)] * 3,
        interpret=_INTERP,
    )(lo, hi, den, gb, lng, lnb, wl, bl, wr, br)


def _post1_body(lo_r, hi_r, den_r, gb_r, lng_r, lnb_r, res_r, h1_r):
    h1_r[...] = _agg_norm(lo_r[...], hi_r[...], den_r[...], gb_r[...],
                          lng_r[...], lnb_r[...]) + res_r[...]


def _post1(lo, hi, den, gb, lng, lnb, res):
    return pl.pallas_call(
        _post1_body,
        grid=(NN // NB,),
        in_specs=[_rows((NB, HW)), _rows((NB, HW)), _rows((NB, 16)),
                  _full((1, HC)), _full((1, HC)), _full((1, HC)), _rows((NB, HC))],
        out_specs=_rows((NB, HC)),
        out_shape=jax.ShapeDtypeStruct((NN, HC), jnp.float32),
        interpret=_INTERP,
    )(lo, hi, den, gb, lng, lnb, res)


def _c1_body(hp_r, h_r, hn_r, pm_r, nm_r, w0_r, w1_r, w2_r, cb_r, g_r, b_r, t1_r):
    y = (_dot(hp_r[...] * pm_r[...], w0_r[...]) + _dot(h_r[...], w1_r[...])
         + _dot(hn_r[...] * nm_r[...], w2_r[...]) + cb_r[...])
    t1_r[...] = jnp.maximum(y * _BN_K * g_r[...] + b_r[...], 0.0)


def _conv1(hp, h, hn, pm, nm, w0, w1, w2, cb, g, b):
    return pl.pallas_call(
        _c1_body,
        grid=(NN // NB,),
        in_specs=[_rows((NB, HC))] * 3 + [_rows((NB, 1))] * 2
                 + [_full((HC, HC))] * 3 + [_full((1, HC))] * 3,
        out_specs=_rows((NB, HC)),
        out_shape=jax.ShapeDtypeStruct((NN, HC), jnp.float32),
        interpret=_INTERP,
    )(hp, h, hn, pm, nm, w0, w1, w2, cb, g, b)


def _c2_body(tp_r, t_r, tn_r, pm_r, nm_r, sel_r, lw2_r, res_r,
             w0_r, w1_r, w2_r, cb_r, g_r, b_r, t2_r):
    y = (_dot(tp_r[...] * pm_r[...], w0_r[...]) + _dot(t_r[...], w1_r[...])
         + _dot(tn_r[...] * nm_r[...], w2_r[...])
         + _dot_hi(sel_r[...], lw2_r[...]) + cb_r[...])
    t2_r[...] = jnp.maximum(y * _BN_K * g_r[...] + b_r[...] + res_r[...], 0.0)


def _conv2(tp, t, tn, pm, nm, sel, lw2, res, w0, w1, w2, cb, g, b):
    return pl.pallas_call(
        _c2_body,
        grid=(NN // NB,),
        in_specs=[_rows((NB, HC))] * 3 + [_rows((NB, 1))] * 2
                 + [_rows((NB, NG)), _full((NG, HC)), _rows((NB, HC))]
                 + [_full((HC, HC))] * 3 + [_full((1, HC))] * 3,
        out_specs=_rows((NB, HC)),
        out_shape=jax.ShapeDtypeStruct((NN, HC), jnp.float32),
        interpret=_INTERP,
    )(tp, t, tn, pm, nm, sel, lw2, res, w0, w1, w2, cb, g, b)


def _leak_body(hl_r, t1l_r, cnt_r, tv_r, w0c1_r, c1b_r, g1_r, b1_r,
               w0c2_r, w1c2_r, w2c2_r, c2b_r, g2_r, b2_r,
               lw2_r, psum_r, pmax_r):
    cb = cnt_r[...]
    tv = tv_r[...]
    l1 = jnp.maximum((_dot(hl_r[...], w0c1_r[...]) + c1b_r[...]) * _BN_K
                     * g1_r[...] + b1_r[...], 0.0)
    m1 = ((cb > 0.0) & (cb < float(NN)) & (cb < tv)).astype(jnp.float32)
    l1m = l1 * m1
    p1 = jnp.maximum((_dot(t1l_r[...], w0c2_r[...]) + _dot(l1m, w1c2_r[...])
                      + c2b_r[...]) * _BN_K * g2_r[...] + b2_r[...], 0.0) * m1
    m2 = ((cb + 1.0 < tv) & (cb + 1.0 < float(NN))).astype(jnp.float32)
    p2 = jnp.maximum((_dot(l1m, w0c2_r[...]) + c2b_r[...]) * _BN_K
                     * g2_r[...] + b2_r[...], 0.0) * m2
    lw2_r[...] = _dot(l1m, w2c2_r[...])
    psum_r[...] = p1 + p2
    pmax_r[...] = jnp.maximum(p1, p2)


def _leak(hl, t1l, cnt, tv, w0c1, c1b, g1, b1, w0c2, w1c2, w2c2, c2b, g2, b2):
    return pl.pallas_call(
        _leak_body,
        in_specs=[_full((NG, HC))] * 2 + [_full((NG, 1)), _full((1, 1)),
                  _full((HC, HC)), _full((1, HC)), _full((1, HC)), _full((1, HC)),
                  _full((HC, HC)), _full((HC, HC)), _full((HC, HC)),
                  _full((1, HC)), _full((1, HC)), _full((1, HC))],
        out_specs=[_full((NG, HC))] * 3,
        out_shape=[jax.ShapeDtypeStruct((NG, HC), jnp.float32)] * 3,
        interpret=_INTERP,
    )(hl, t1l, cnt, tv, w0c1, c1b, g1, b1, w0c2, w1c2, w2c2, c2b, g2, b2)


def _pool_body(t2_r, oht_r, oh_r, psum_r, pmax_r, tv_r,
               w1_r, b1_r, w2_r, b2_r, w3_r, b3_r, out_r):
    t2 = t2_r[...]
    sums = _dot_hi(oht_r[...], t2)
    oh = oh_r[...]
    maxs = [jnp.max(t2 * oh[:, b:b + 1], axis=0, keepdims=True)
            for b in range(NG)]
    mx = jnp.concatenate(maxs, axis=0)
    mean = (sums + psum_r[...]) / tv_r[...]
    mx = jnp.maximum(mx, pmax_r[...])
    g = jnp.concatenate([mean, mx], axis=1)
    g = jnp.maximum(_dot(g, w1_r[...]) + b1_r[...], 0.0)
    g = jnp.maximum(_dot(g, w2_r[...]) + b2_r[...], 0.0)
    out_r[...] = _dot(g, w3_r[...]) + b3_r[...]


def _pool(t2, oht, oh, psum, pmax, tv, w1, b1, w2, b2, w3, b3):
    return pl.pallas_call(
        _pool_body,
        in_specs=[_full((NN, HC)), _full((NG, NN)), _full((NN, NG)),
                  _full((NG, HC)), _full((NG, HC)), _full((1, 1)),
                  _full((2 * HC, HC)), _full((1, HC)),
                  _full((HC, DH)), _full((1, DH)),
                  _full((DH, 1)), _full((1, 1))],
        out_specs=_full((NG, 1)),
        out_shape=jax.ShapeDtypeStruct((NG, 1), jnp.float32),
        interpret=_INTERP,
    )(t2, oht, oh, psum, pmax, tv, w1, b1, w2, b2, w3, b3)


_NC = 2
_NS = 16
CH = 80
HW = HC // 2
NNP = 10240
RPT = NNP // _NS


def _sc_mesh():
    return plsc.VectorSubcoreMesh(core_axis_name="c", subcore_axis_name="s")


def _qadd_body(xl_hbm, xr_hbm, src_hbm, dst_hbm, q_hbm, sidx, didx, xlb, xrb):
    c = lax.axis_index("c")
    s = lax.axis_index("s")
    wid = s * _NC + c
    nper = NE // (_NC * _NS)
    base0 = wid * nper

    def chunk(k, carry):
        base = base0 + k * CH
        pltpu.sync_copy(src_hbm.at[pl.ds(base, CH)], sidx)
        pltpu.sync_copy(dst_hbm.at[pl.ds(base, CH)], didx)
        pltpu.sync_copy(xl_hbm.at[sidx], xlb)
        pltpu.sync_copy(xr_hbm.at[didx], xrb)

        def edge(e, cc):
            for j in range(HC // 16):
                xlb[e, pl.ds(16 * j, 16)] = (xlb[e, pl.ds(16 * j, 16)]
                                             + xrb[e, pl.ds(16 * j, 16)])
            return cc
        lax.fori_loop(0, CH, edge, 0)
        pltpu.sync_copy(xlb, q_hbm.at[pl.ds(base, CH)])
        return carry
    lax.fori_loop(0, nper // CH, chunk, 0)


def _qadd(xl, xr, src, dst):
    f = pl.kernel(
        _qadd_body, mesh=_sc_mesh(),
        out_type=jax.ShapeDtypeStruct((NE, HC), jnp.float32),
        scratch_types=[pltpu.VMEM((CH,), jnp.int32), pltpu.VMEM((CH,), jnp.int32),
                       pltpu.VMEM((CH, HC), jnp.float32),
                       pltpu.VMEM((CH, HC), jnp.float32)],
    )
    return f(xl, xr, src, dst)


NDR = NNP // 8


def _scat_body(xl2_hbm, src_hbm, dst_hbm, sflat_hbm, zin_hbm, lo_hbm, hi_hbm,
               dfl_hbm, acc, dtab, sidx, didx, didx3, idx2, sbuf, xlb, cbuf, cbuf2):
    c = lax.axis_index("c")
    s = lax.axis_index("s")
    pltpu.sync_copy(zin_hbm, acc.at[pl.ds(s * RPT, RPT)])
    pltpu.sync_copy(zin_hbm.at[pl.ds(0, NDR // _NS)],
                    dtab.at[pl.ds(s * (NDR // _NS), NDR // _NS)])
    lane = jnp.arange(16, dtype=jnp.int32)
    zero16 = jnp.zeros((16,), jnp.float32)
    plsc.subcore_barrier()

    nper = NE // _NS
    base0 = s * nper

    def chunk(k, carry):
        base = base0 + k * CH
        pltpu.sync_copy(src_hbm.at[pl.ds(base, CH)], sidx)
        pltpu.sync_copy(dst_hbm.at[pl.ds(base, CH)], didx)
        pltpu.sync_copy(sflat_hbm.at[pl.ds(NH * base, NH * CH)],
                        sbuf.at[pl.ds(0, NH * CH)])

        def vec(v, cc):
            dv = didx[pl.ds(16 * v, 16)]
            idx2[pl.ds(16 * v, 16)] = sidx[pl.ds(16 * v, 16)] * 2 + c
            didx3[pl.ds(16 * v, 16)] = lax.shift_right_logical(dv, 3)
            return cc
        lax.fori_loop(0, CH // 16, vec, 0)
        pltpu.sync_copy(xl2_hbm.at[idx2], xlb)

        def grp(g, cc):
            dvec = didx[pl.ds(16 * g, 16)]
            for i in range(16):
                e = 16 * g + i
                sv = sbuf[pl.ds(NH * e, 16)]
                sp0 = sv[jnp.broadcast_to(2 * c, (16,))]
                sp1 = sv[jnp.broadcast_to(2 * c + 1, (16,))]
                for j in range(HW // 16):
                    sp = sp0 if j < (HW // 32) else sp1
                    cbuf[e, pl.ds(16 * j, 16)] = sp * xlb[e, pl.ds(16 * j, 16)]
                for j in range(8):
                    cbuf2[e, pl.ds(16 * j, 16)] = zero16
                off = 16 * (dvec[i] & 7)
                cbuf2[e, pl.ds(off, 16)] = jnp.where(lane < NH, sv, 0.0)
            return cc
        lax.fori_loop(0, CH // 16, grp, 0)
        pltpu.sync_copy(cbuf, acc.at[didx], add=True)
        pltpu.sync_copy(cbuf2, dtab.at[didx3], add=True)
        return carry
    lax.fori_loop(0, nper // CH, chunk, 0)
    plsc.subcore_barrier()

    @pl.when(c == 0)
    def _():
        pltpu.sync_copy(acc.at[pl.ds(s * RPT, RPT)], lo_hbm.at[pl.ds(s * RPT, RPT)])
        pltpu.sync_copy(dtab.at[pl.ds(s * (NDR // _NS), NDR // _NS)],
                        dfl_hbm.at[pl.ds(s * (NDR // _NS), NDR // _NS)])

    @pl.when(c == 1)
    def _():
        pltpu.sync_copy(acc.at[pl.ds(s * RPT, RPT)], hi_hbm.at[pl.ds(s * RPT, RPT)])


def _scatter(xl2, src, dst, s_flat, zin):
    f = pl.kernel(
        _scat_body, mesh=_sc_mesh(),
        out_type=[jax.ShapeDtypeStruct((NNP, HW), jnp.float32),
                  jax.ShapeDtypeStruct((NNP, HW), jnp.float32),
                  jax.ShapeDtypeStruct((NDR, 128), jnp.float32)],
        scratch_types=[pltpu.VMEM_SHARED((NNP, HW), jnp.float32),
                       pltpu.VMEM_SHARED((NDR, 128), jnp.float32),
                       pltpu.VMEM((CH,), jnp.int32), pltpu.VMEM((CH,), jnp.int32),
                       pltpu.VMEM((CH,), jnp.int32), pltpu.VMEM((CH,), jnp.int32),
                       pltpu.VMEM((NH * CH + 16,), jnp.float32),
                       pltpu.VMEM((CH, HW), jnp.float32),
                       pltpu.VMEM((CH, HW), jnp.float32),
                       pltpu.VMEM((CH, 128), jnp.float32)],
    )
    return f(xl2, src, dst, s_flat, zin)


def _edge_aggregate(xl, xr, ea, src, dst, we, att_flat, zin):
    q = _qadd(xl, xr, src, dst)
    s = _alpha(q, ea, we, att_flat)
    lo, hi, dfl = _scatter(xl.reshape(2 * NN, HW), src, dst,
                           s.reshape(NE * NH), zin)
    den16 = dfl.reshape(NNP, 16)
    return lo[:NN], hi[:NN], den16[:NN]


def kernel(x, edge_index, edge_attr, batch, params):
    p = params
    src = edge_index[0]
    dst = edge_index[1]
    r1 = lambda a: a.reshape(1, -1)

    zin = jnp.zeros((RPT, HW), jnp.float32)

    xl0, xr0 = _pre(x, p['Win'], r1(p['bin']), p['g0_Wl'], r1(p['g0_bl']),
                    p['g0_Wr'], r1(p['g0_br']))
    lo0, hi0, den0 = _edge_aggregate(xl0, xr0, edge_attr, src, dst,
                                     p['g0_We'], r1(p['g0_att']), zin)
    h0, xl1, xr1 = _mid(lo0, hi0, den0, r1(p['g0_bias']), r1(p['ln0_g']),
                        r1(p['ln0_b']), p['g1_Wl'], r1(p['g1_bl']),
                        p['g1_Wr'], r1(p['g1_br']))
    lo1, hi1, den1 = _edge_aggregate(xl1, xr1, edge_attr, src, dst,
                                     p['g1_We'], r1(p['g1_att']), zin)
    h1 = _post1(lo1, hi1, den1, r1(p['g1_bias']), r1(p['ln1_g']),
                r1(p['ln1_b']), h0)

    idx = jnp.arange(NN, dtype=jnp.int32)
    o = jnp.searchsorted(batch, jnp.arange(NG + 1, dtype=jnp.int32)).astype(jnp.int32)
    cnt = (o[1:] - o[:-1]).astype(jnp.float32)
    tv = jnp.max(cnt)
    same_next = (batch[1:] == batch[:-1])
    pm = jnp.concatenate([jnp.zeros((1,), jnp.bool_), same_next]).astype(jnp.float32)[:, None]
    nm = jnp.concatenate([same_next, jnp.zeros((1,), jnp.bool_)]).astype(jnp.float32)[:, None]
    z1 = jnp.zeros((1, HC), jnp.float32)
    shift = lambda a: (jnp.concatenate([z1, a[:-1]], 0), jnp.concatenate([a[1:], z1], 0))

    lastrow = jnp.clip(o[1:] - 1, 0, NN - 1)
    sel = (idx[:, None] == (o[1:] - 1)[None, :]).astype(jnp.float32)
    oh = (batch[:, None] == jnp.arange(NG, dtype=jnp.int32)[None, :]).astype(jnp.float32)
    oht = oh.T

    wc1 = [p['c1_w'][:, :, k].T for k in range(3)]
    wc2 = [p['c2_w'][:, :, k].T for k in range(3)]

    h1p, h1n = shift(h1)
    t1 = _conv1(h1p, h1, h1n, pm, nm, wc1[0], wc1[1], wc1[2],
                r1(p['c1_b']), r1(p['bn1_g']), r1(p['bn1_b']))

    hl = h1[lastrow]
    t1l = t1[lastrow]
    lw2, psum, pmax = _leak(hl, t1l, cnt[:, None], tv.reshape(1, 1),
                            wc1[0], r1(p['c1_b']), r1(p['bn1_g']), r1(p['bn1_b']),
                            wc2[0], wc2[1], wc2[2],
                            r1(p['c2_b']), r1(p['bn2_g']), r1(p['bn2_b']))

    t1p, t1n = shift(t1)
    t2 = _conv2(t1p, t1, t1n, pm, nm, sel, lw2, h1,
                wc2[0], wc2[1], wc2[2],
                r1(p['c2_b']), r1(p['bn2_g']), r1(p['bn2_b']))

    out = _pool(t2, oht, oh, psum, pmax, tv.reshape(1, 1),
                p['w1'], r1(p['b1']), p['w2'], r1(p['b2']),
                p['w3'], r1(p['b3']))
    return out.reshape(NG)


# speedup vs baseline: 19.4943x; 1.2033x over previous
import functools

import jax
import jax.numpy as jnp
from jax import lax
from jax.experimental import pallas as pl
from jax.experimental.pallas import tpu as pltpu
from jax.experimental.pallas import tpu_sc as plsc

NN = 10000
NE = 320000
DIN = 128
DE = 4
NH = 4
DH = 64
HC = 256
NG = 8
_BN_K = 1.0 / (1.0 + 1e-5) ** 0.5

_INTERP = False

NB = 1000
EB = 4000


def _dot(a, b):
    return jnp.dot(a, b, preferred_element_type=jnp.float32)


def _dot_hi(a, b):
    return jnp.dot(a, b, preferred_element_type=jnp.float32,
                   precision=lax.Precision.HIGHEST)


def _full(shape):
    return pl.BlockSpec(shape, lambda *a: tuple(0 for _ in shape))


def _rows(shape):
    return pl.BlockSpec(shape, lambda i: (i,) + tuple(0 for _ in shape[1:]))


def _pre_body(x_r, win_r, bin_r, wl_r, bl_r, wr_r, br_r, xl_r, xr_r):
    h = _dot(x_r[...], win_r[...]) + bin_r[...]
    xl_r[...] = _dot(h, wl_r[...]) + bl_r[...]
    xr_r[...] = _dot(h, wr_r[...]) + br_r[...]


def _pre(x, win, bin_, wl, bl, wr, br):
    return pl.pallas_call(
        _pre_body,
        grid=(NN // NB,),
        in_specs=[_rows((NB, DIN)), _full((DIN, DH)), _full((1, DH)),
                  _full((DH, HC)), _full((1, HC)), _full((DH, HC)), _full((1, HC))],
        out_specs=[_rows((NB, HC)), _rows((NB, HC))],
        out_shape=[jax.ShapeDtypeStruct((NN, HC), jnp.float32)] * 2,
        interpret=_INTERP,
    )(x, win, bin_, wl, bl, wr, br)


def _alpha_body(q_r, ea_r, we_r, att_r, s_r):
    m = q_r[...] + _dot(ea_r[...], we_r[...])
    m = jnp.where(m >= 0.0, m, 0.2 * m)
    ma = m * att_r[...]
    parts = [jnp.sum(ma[:, h * DH:(h + 1) * DH], axis=1, keepdims=True)
             for h in range(NH)]
    s_r[...] = jnp.exp(jnp.concatenate(parts, axis=1))


def _alpha(q, ea, we, att_flat):
    return pl.pallas_call(
        _alpha_body,
        grid=(NE // EB,),
        in_specs=[_rows((EB, HC)), _rows((EB, DE)), _full((DE, HC)), _full((1, HC))],
        out_specs=_rows((EB, NH)),
        out_shape=jax.ShapeDtypeStruct((NE, NH), jnp.float32),
        interpret=_INTERP,
    )(q, ea, we, att_flat)


def _norm(acc, den256, gb, lng, lnb):
    o = acc / (den256 + 1e-16) + gb
    mu = jnp.mean(o, axis=1, keepdims=True)
    v = jnp.mean((o - mu) ** 2, axis=1, keepdims=True)
    o = (o - mu) / jnp.sqrt(v + 1e-5) * lng + lnb
    return jnp.where(o > 0.0, o, jnp.exp(o) - 1.0)


def _agg_norm(lo, hi, den16, gb, lng, lnb):
    acc = jnp.concatenate([lo, hi], axis=1)
    o = jnp.concatenate(
        [acc[:, h * DH:(h + 1) * DH] / (den16[:, h:h + 1] + 1e-16)
         for h in range(NH)], axis=1) + gb
    mu = jnp.mean(o, axis=1, keepdims=True)
    v = jnp.mean((o - mu) ** 2, axis=1, keepdims=True)
    o = (o - mu) / jnp.sqrt(v + 1e-5) * lng + lnb
    return jnp.where(o > 0.0, o, jnp.exp(o) - 1.0)


def _mid_body(lo_r, hi_r, den_r, gb_r, lng_r, lnb_r, wl_r, bl_r, wr_r, br_r,
              h0_r, xl_r, xr_r):
    h0 = _agg_norm(lo_r[...], hi_r[...], den_r[...], gb_r[...],
                   lng_r[...], lnb_r[...])
    h0_r[...] = h0
    xl_r[...] = _dot(h0, wl_r[...]) + bl_r[...]
    xr_r[...] = _dot(h0, wr_r[...]) + br_r[...]


def _mid(lo, hi, den, gb, lng, lnb, wl, bl, wr, br):
    return pl.pallas_call(
        _mid_body,
        grid=(NN // NB,),
        in_specs=[_rows((NB, HW)), _rows((NB, HW)), _rows((NB, 16)),
                  _full((1, HC)), _full((1, HC)), _full((1, HC)),
                  _full((HC, HC)), _full((1, HC)), _full((HC, HC)), _full((1, HC))],
        out_specs=[_rows((NB, HC))] * 3,
        out_shape=[jax.ShapeDtypeStruct((NN, HC), jnp.float32)] * 3,
        interpret=_INTERP,
    )(lo, hi, den, gb, lng, lnb, wl, bl, wr, br)


def _post1_body(lo_r, hi_r, den_r, gb_r, lng_r, lnb_r, res_r, h1_r):
    h1_r[...] = _agg_norm(lo_r[...], hi_r[...], den_r[...], gb_r[...],
                          lng_r[...], lnb_r[...]) + res_r[...]


def _post1(lo, hi, den, gb, lng, lnb, res):
    return pl.pallas_call(
        _post1_body,
        grid=(NN // NB,),
        in_specs=[_rows((NB, HW)), _rows((NB, HW)), _rows((NB, 16)),
                  _full((1, HC)), _full((1, HC)), _full((1, HC)), _rows((NB, HC))],
        out_specs=_rows((NB, HC)),
        out_shape=jax.ShapeDtypeStruct((NN, HC), jnp.float32),
        interpret=_INTERP,
    )(lo, hi, den, gb, lng, lnb, res)


def _c1_body(hp_r, h_r, hn_r, pm_r, nm_r, w0_r, w1_r, w2_r, cb_r, g_r, b_r, t1_r):
    y = (_dot(hp_r[...] * pm_r[...], w0_r[...]) + _dot(h_r[...], w1_r[...])
         + _dot(hn_r[...] * nm_r[...], w2_r[...]) + cb_r[...])
    t1_r[...] = jnp.maximum(y * _BN_K * g_r[...] + b_r[...], 0.0)


def _conv1(hp, h, hn, pm, nm, w0, w1, w2, cb, g, b):
    return pl.pallas_call(
        _c1_body,
        grid=(NN // NB,),
        in_specs=[_rows((NB, HC))] * 3 + [_rows((NB, 1))] * 2
                 + [_full((HC, HC))] * 3 + [_full((1, HC))] * 3,
        out_specs=_rows((NB, HC)),
        out_shape=jax.ShapeDtypeStruct((NN, HC), jnp.float32),
        interpret=_INTERP,
    )(hp, h, hn, pm, nm, w0, w1, w2, cb, g, b)


def _c2_body(tp_r, t_r, tn_r, pm_r, nm_r, sel_r, lw2_r, res_r,
             w0_r, w1_r, w2_r, cb_r, g_r, b_r, t2_r):
    y = (_dot(tp_r[...] * pm_r[...], w0_r[...]) + _dot(t_r[...], w1_r[...])
         + _dot(tn_r[...] * nm_r[...], w2_r[...])
         + _dot_hi(sel_r[...], lw2_r[...]) + cb_r[...])
    t2_r[...] = jnp.maximum(y * _BN_K * g_r[...] + b_r[...] + res_r[...], 0.0)


def _conv2(tp, t, tn, pm, nm, sel, lw2, res, w0, w1, w2, cb, g, b):
    return pl.pallas_call(
        _c2_body,
        grid=(NN // NB,),
        in_specs=[_rows((NB, HC))] * 3 + [_rows((NB, 1))] * 2
                 + [_rows((NB, NG)), _full((NG, HC)), _rows((NB, HC))]
                 + [_full((HC, HC))] * 3 + [_full((1, HC))] * 3,
        out_specs=_rows((NB, HC)),
        out_shape=jax.ShapeDtypeStruct((NN, HC), jnp.float32),
        interpret=_INTERP,
    )(tp, t, tn, pm, nm, sel, lw2, res, w0, w1, w2, cb, g, b)


def _leak_body(hl_r, t1l_r, cnt_r, tv_r, w0c1_r, c1b_r, g1_r, b1_r,
               w0c2_r, w1c2_r, w2c2_r, c2b_r, g2_r, b2_r,
               lw2_r, psum_r, pmax_r):
    cb = cnt_r[...]
    tv = tv_r[...]
    l1 = jnp.maximum((_dot(hl_r[...], w0c1_r[...]) + c1b_r[...]) * _BN_K
                     * g1_r[...] + b1_r[...], 0.0)
    m1 = ((cb > 0.0) & (cb < float(NN)) & (cb < tv)).astype(jnp.float32)
    l1m = l1 * m1
    p1 = jnp.maximum((_dot(t1l_r[...], w0c2_r[...]) + _dot(l1m, w1c2_r[...])
                      + c2b_r[...]) * _BN_K * g2_r[...] + b2_r[...], 0.0) * m1
    m2 = ((cb + 1.0 < tv) & (cb + 1.0 < float(NN))).astype(jnp.float32)
    p2 = jnp.maximum((_dot(l1m, w0c2_r[...]) + c2b_r[...]) * _BN_K
                     * g2_r[...] + b2_r[...], 0.0) * m2
    lw2_r[...] = _dot(l1m, w2c2_r[...])
    psum_r[...] = p1 + p2
    pmax_r[...] = jnp.maximum(p1, p2)


def _leak(hl, t1l, cnt, tv, w0c1, c1b, g1, b1, w0c2, w1c2, w2c2, c2b, g2, b2):
    return pl.pallas_call(
        _leak_body,
        in_specs=[_full((NG, HC))] * 2 + [_full((NG, 1)), _full((1, 1)),
                  _full((HC, HC)), _full((1, HC)), _full((1, HC)), _full((1, HC)),
                  _full((HC, HC)), _full((HC, HC)), _full((HC, HC)),
                  _full((1, HC)), _full((1, HC)), _full((1, HC))],
        out_specs=[_full((NG, HC))] * 3,
        out_shape=[jax.ShapeDtypeStruct((NG, HC), jnp.float32)] * 3,
        interpret=_INTERP,
    )(hl, t1l, cnt, tv, w0c1, c1b, g1, b1, w0c2, w1c2, w2c2, c2b, g2, b2)


def _pool_body(t2_r, oht_r, oh_r, psum_r, pmax_r, tv_r,
               w1_r, b1_r, w2_r, b2_r, w3_r, b3_r, out_r):
    t2 = t2_r[...]
    sums = _dot_hi(oht_r[...], t2)
    oh = oh_r[...]
    maxs = [jnp.max(t2 * oh[:, b:b + 1], axis=0, keepdims=True)
            for b in range(NG)]
    mx = jnp.concatenate(maxs, axis=0)
    mean = (sums + psum_r[...]) / tv_r[...]
    mx = jnp.maximum(mx, pmax_r[...])
    g = jnp.concatenate([mean, mx], axis=1)
    g = jnp.maximum(_dot(g, w1_r[...]) + b1_r[...], 0.0)
    g = jnp.maximum(_dot(g, w2_r[...]) + b2_r[...], 0.0)
    out_r[...] = _dot(g, w3_r[...]) + b3_r[...]


def _pool(t2, oht, oh, psum, pmax, tv, w1, b1, w2, b2, w3, b3):
    return pl.pallas_call(
        _pool_body,
        in_specs=[_full((NN, HC)), _full((NG, NN)), _full((NN, NG)),
                  _full((NG, HC)), _full((NG, HC)), _full((1, 1)),
                  _full((2 * HC, HC)), _full((1, HC)),
                  _full((HC, DH)), _full((1, DH)),
                  _full((DH, 1)), _full((1, 1))],
        out_specs=_full((NG, 1)),
        out_shape=jax.ShapeDtypeStruct((NG, 1), jnp.float32),
        interpret=_INTERP,
    )(t2, oht, oh, psum, pmax, tv, w1, b1, w2, b2, w3, b3)


_NC = 2
_NS = 16
CH = 80
HW = HC // 2
NNP = 10240
RPT = NNP // _NS


def _sc_mesh():
    return plsc.VectorSubcoreMesh(core_axis_name="c", subcore_axis_name="s")


QCH = 40


def _qadd_body(xl_hbm, xr_hbm, src_hbm, dst_hbm, q_hbm,
               sidx0, didx0, sidx1, didx1, xlb0, xrb0, wb0, xlb1, xrb1, wb1,
               gl0, gr0, gl1, gr1, w0, w1, is0, is1):
    c = lax.axis_index("c")
    s = lax.axis_index("s")
    wid = s * _NC + c
    nper = NE // (_NC * _NS)
    base0 = wid * nper
    nch = nper // QCH
    sidx = [sidx0, sidx1]
    didx = [didx0, didx1]
    xlb = [xlb0, xlb1]
    xrb = [xrb0, xrb1]
    wb = [wb0, wb1]
    gl = [gl0, gl1]
    gr = [gr0, gr1]
    ws = [w0, w1]
    iss = [is0, is1]

    def load_idx(k, p):
        base = base0 + k * QCH
        pltpu.async_copy(src_hbm.at[pl.ds(base, QCH)], sidx[p], iss[p])
        pltpu.async_copy(dst_hbm.at[pl.ds(base, QCH)], didx[p], iss[p])

    def wait_idx(p):
        pltpu.make_async_copy(src_hbm.at[pl.ds(0, QCH)], sidx[p], iss[p]).wait()
        pltpu.make_async_copy(dst_hbm.at[pl.ds(0, QCH)], didx[p], iss[p]).wait()

    def gather(p):
        pltpu.async_copy(xl_hbm.at[sidx[p]], xlb[p], gl[p])
        pltpu.async_copy(xr_hbm.at[didx[p]], xrb[p], gr[p])

    def wait_gather(p):
        pltpu.make_async_copy(xl_hbm.at[sidx[p]], xlb[p], gl[p]).wait()
        pltpu.make_async_copy(xr_hbm.at[didx[p]], xrb[p], gr[p]).wait()

    load_idx(0, 0)
    wait_idx(0)
    gather(0)
    load_idx(1, 1)
    wait_idx(1)
    gather(1)

    def body(m, carry):
        for p in range(2):
            k = 2 * m + p
            wait_gather(p)

            @pl.when(k + 2 < nch)
            def _():
                load_idx(k + 2, p)

            @pl.when(k >= 2)
            def _():
                pltpu.make_async_copy(wb[p], q_hbm.at[pl.ds(0, QCH)], ws[p]).wait()

            def edge(e, cc):
                for j in range(HC // 16):
                    wb[p][e, pl.ds(16 * j, 16)] = (xlb[p][e, pl.ds(16 * j, 16)]
                                                   + xrb[p][e, pl.ds(16 * j, 16)])
                return cc
            lax.fori_loop(0, QCH, edge, 0)
            pltpu.async_copy(wb[p], q_hbm.at[pl.ds(base0 + k * QCH, QCH)], ws[p])

            @pl.when(k + 2 < nch)
            def _():
                wait_idx(p)
                gather(p)
        return carry
    lax.fori_loop(0, nch // 2, body, 0)
    pltpu.make_async_copy(wb[0], q_hbm.at[pl.ds(0, QCH)], ws[0]).wait()
    pltpu.make_async_copy(wb[1], q_hbm.at[pl.ds(0, QCH)], ws[1]).wait()


def _qadd(xl, xr, src, dst):
    f = pl.kernel(
        _qadd_body, mesh=_sc_mesh(),
        out_type=jax.ShapeDtypeStruct((NE, HC), jnp.float32),
        scratch_types=[pltpu.VMEM((QCH,), jnp.int32), pltpu.VMEM((QCH,), jnp.int32),
                       pltpu.VMEM((QCH,), jnp.int32), pltpu.VMEM((QCH,), jnp.int32),
                       pltpu.VMEM((QCH, HC), jnp.float32),
                       pltpu.VMEM((QCH, HC), jnp.float32),
                       pltpu.VMEM((QCH, HC), jnp.float32),
                       pltpu.VMEM((QCH, HC), jnp.float32),
                       pltpu.VMEM((QCH, HC), jnp.float32),
                       pltpu.VMEM((QCH, HC), jnp.float32)]
                      + [pltpu.SemaphoreType.DMA] * 8,
    )
    return f(xl, xr, src, dst)


NDR = NNP // 8


def _scat_body(xl2_hbm, src_hbm, dst_hbm, sflat_hbm, zin_hbm, lo_hbm, hi_hbm,
               dfl_hbm, acc, dtab, sidx, didx, didx3, idx2, sbuf, xlb, cbuf, cbuf2):
    c = lax.axis_index("c")
    s = lax.axis_index("s")
    pltpu.sync_copy(zin_hbm, acc.at[pl.ds(s * RPT, RPT)])
    pltpu.sync_copy(zin_hbm.at[pl.ds(0, NDR // _NS)],
                    dtab.at[pl.ds(s * (NDR // _NS), NDR // _NS)])
    lane = jnp.arange(16, dtype=jnp.int32)
    zero16 = jnp.zeros((16,), jnp.float32)
    plsc.subcore_barrier()

    nper = NE // _NS
    base0 = s * nper

    def chunk(k, carry):
        base = base0 + k * CH
        pltpu.sync_copy(src_hbm.at[pl.ds(base, CH)], sidx)
        pltpu.sync_copy(dst_hbm.at[pl.ds(base, CH)], didx)
        pltpu.sync_copy(sflat_hbm.at[pl.ds(NH * base, NH * CH)],
                        sbuf.at[pl.ds(0, NH * CH)])

        def vec(v, cc):
            dv = didx[pl.ds(16 * v, 16)]
            idx2[pl.ds(16 * v, 16)] = sidx[pl.ds(16 * v, 16)] * 2 + c
            didx3[pl.ds(16 * v, 16)] = lax.shift_right_logical(dv, 3)
            return cc
        lax.fori_loop(0, CH // 16, vec, 0)
        pltpu.sync_copy(xl2_hbm.at[idx2], xlb)

        def grp(g, cc):
            dvec = didx[pl.ds(16 * g, 16)]
            for i in range(16):
                e = 16 * g + i
                sv = sbuf[pl.ds(NH * e, 16)]
                sp0 = sv[jnp.broadcast_to(2 * c, (16,))]
                sp1 = sv[jnp.broadcast_to(2 * c + 1, (16,))]
                for j in range(HW // 16):
                    sp = sp0 if j < (HW // 32) else sp1
                    cbuf[e, pl.ds(16 * j, 16)] = sp * xlb[e, pl.ds(16 * j, 16)]
                for j in range(8):
                    cbuf2[e, pl.ds(16 * j, 16)] = zero16
                off = 16 * (dvec[i] & 7)
                cbuf2[e, pl.ds(off, 16)] = jnp.where(lane < NH, sv, 0.0)
            return cc
        lax.fori_loop(0, CH // 16, grp, 0)
        pltpu.sync_copy(cbuf, acc.at[didx], add=True)
        pltpu.sync_copy(cbuf2, dtab.at[didx3], add=True)
        return carry
    lax.fori_loop(0, nper // CH, chunk, 0)
    plsc.subcore_barrier()

    @pl.when(c == 0)
    def _():
        pltpu.sync_copy(acc.at[pl.ds(s * RPT, RPT)], lo_hbm.at[pl.ds(s * RPT, RPT)])
        pltpu.sync_copy(dtab.at[pl.ds(s * (NDR // _NS), NDR // _NS)],
                        dfl_hbm.at[pl.ds(s * (NDR // _NS), NDR // _NS)])

    @pl.when(c == 1)
    def _():
        pltpu.sync_copy(acc.at[pl.ds(s * RPT, RPT)], hi_hbm.at[pl.ds(s * RPT, RPT)])


def _scatter(xl2, src, dst, s_flat, zin):
    f = pl.kernel(
        _scat_body, mesh=_sc_mesh(),
        out_type=[jax.ShapeDtypeStruct((NNP, HW), jnp.float32),
                  jax.ShapeDtypeStruct((NNP, HW), jnp.float32),
                  jax.ShapeDtypeStruct((NDR, 128), jnp.float32)],
        scratch_types=[pltpu.VMEM_SHARED((NNP, HW), jnp.float32),
                       pltpu.VMEM_SHARED((NDR, 128), jnp.float32),
                       pltpu.VMEM((CH,), jnp.int32), pltpu.VMEM((CH,), jnp.int32),
                       pltpu.VMEM((CH,), jnp.int32), pltpu.VMEM((CH,), jnp.int32),
                       pltpu.VMEM((NH * CH + 16,), jnp.float32),
                       pltpu.VMEM((CH, HW), jnp.float32),
                       pltpu.VMEM((CH, HW), jnp.float32),
                       pltpu.VMEM((CH, 128), jnp.float32)],
    )
    return f(xl2, src, dst, s_flat, zin)


def _edge_aggregate(xl, xr, ea, src, dst, we, att_flat, zin):
    q = _qadd(xl, xr, src, dst)
    s = _alpha(q, ea, we, att_flat)
    lo, hi, dfl = _scatter(xl.reshape(2 * NN, HW), src, dst,
                           s.reshape(NE * NH), zin)
    den16 = dfl.reshape(NNP, 16)
    return lo[:NN], hi[:NN], den16[:NN]


def kernel(x, edge_index, edge_attr, batch, params):
    p = params
    src = edge_index[0]
    dst = edge_index[1]
    r1 = lambda a: a.reshape(1, -1)

    zin = jnp.zeros((RPT, HW), jnp.float32)

    xl0, xr0 = _pre(x, p['Win'], r1(p['bin']), p['g0_Wl'], r1(p['g0_bl']),
                    p['g0_Wr'], r1(p['g0_br']))
    lo0, hi0, den0 = _edge_aggregate(xl0, xr0, edge_attr, src, dst,
                                     p['g0_We'], r1(p['g0_att']), zin)
    h0, xl1, xr1 = _mid(lo0, hi0, den0, r1(p['g0_bias']), r1(p['ln0_g']),
                        r1(p['ln0_b']), p['g1_Wl'], r1(p['g1_bl']),
                        p['g1_Wr'], r1(p['g1_br']))
    lo1, hi1, den1 = _edge_aggregate(xl1, xr1, edge_attr, src, dst,
                                     p['g1_We'], r1(p['g1_att']), zin)
    h1 = _post1(lo1, hi1, den1, r1(p['g1_bias']), r1(p['ln1_g']),
                r1(p['ln1_b']), h0)

    idx = jnp.arange(NN, dtype=jnp.int32)
    o = jnp.searchsorted(batch, jnp.arange(NG + 1, dtype=jnp.int32)).astype(jnp.int32)
    cnt = (o[1:] - o[:-1]).astype(jnp.float32)
    tv = jnp.max(cnt)
    same_next = (batch[1:] == batch[:-1])
    pm = jnp.concatenate([jnp.zeros((1,), jnp.bool_), same_next]).astype(jnp.float32)[:, None]
    nm = jnp.concatenate([same_next, jnp.zeros((1,), jnp.bool_)]).astype(jnp.float32)[:, None]
    z1 = jnp.zeros((1, HC), jnp.float32)
    shift = lambda a: (jnp.concatenate([z1, a[:-1]], 0), jnp.concatenate([a[1:], z1], 0))

    lastrow = jnp.clip(o[1:] - 1, 0, NN - 1)
    sel = (idx[:, None] == (o[1:] - 1)[None, :]).astype(jnp.float32)
    oh = (batch[:, None] == jnp.arange(NG, dtype=jnp.int32)[None, :]).astype(jnp.float32)
    oht = oh.T

    wc1 = [p['c1_w'][:, :, k].T for k in range(3)]
    wc2 = [p['c2_w'][:, :, k].T for k in range(3)]

    h1p, h1n = shift(h1)
    t1 = _conv1(h1p, h1, h1n, pm, nm, wc1[0], wc1[1], wc1[2],
                r1(p['c1_b']), r1(p['bn1_g']), r1(p['bn1_b']))

    hl = h1[lastrow]
    t1l = t1[lastrow]
    lw2, psum, pmax = _leak(hl, t1l, cnt[:, None], tv.reshape(1, 1),
                            wc1[0], r1(p['c1_b']), r1(p['bn1_g']), r1(p['bn1_b']),
                            wc2[0], wc2[1], wc2[2],
                            r1(p['c2_b']), r1(p['bn2_g']), r1(p['bn2_b']))

    t1p, t1n = shift(t1)
    t2 = _conv2(t1p, t1, t1n, pm, nm, sel, lw2, h1,
                wc2[0], wc2[1], wc2[2],
                r1(p['c2_b']), r1(p['bn2_g']), r1(p['bn2_b']))

    out = _pool(t2, oht, oh, psum, pmax, tv.reshape(1, 1),
                p['w1'], r1(p['b1']), p['w2'], r1(p['b2']),
                p['w3'], r1(p['b3']))
    return out.reshape(NG)


# speedup vs baseline: 21.0602x; 1.0803x over previous
import functools

import jax
import jax.numpy as jnp
from jax import lax
from jax.experimental import pallas as pl
from jax.experimental.pallas import tpu as pltpu
from jax.experimental.pallas import tpu_sc as plsc

NN = 10000
NE = 320000
DIN = 128
DE = 4
NH = 4
DH = 64
HC = 256
NG = 8
_BN_K = 1.0 / (1.0 + 1e-5) ** 0.5

_INTERP = False

NB = 1000
EB = 4000


def _dot(a, b):
    return jnp.dot(a, b, preferred_element_type=jnp.float32)


def _dot_hi(a, b):
    return jnp.dot(a, b, preferred_element_type=jnp.float32,
                   precision=lax.Precision.HIGHEST)


def _full(shape):
    return pl.BlockSpec(shape, lambda *a: tuple(0 for _ in shape))


def _rows(shape):
    return pl.BlockSpec(shape, lambda i: (i,) + tuple(0 for _ in shape[1:]))


def _pre_body(x_r, win_r, bin_r, wl_r, bl_r, wr_r, br_r, xl_r, xr_r):
    h = _dot(x_r[...], win_r[...]) + bin_r[...]
    xl_r[...] = _dot(h, wl_r[...]) + bl_r[...]
    xr_r[...] = _dot(h, wr_r[...]) + br_r[...]


def _pre(x, win, bin_, wl, bl, wr, br):
    return pl.pallas_call(
        _pre_body,
        grid=(NN // NB,),
        in_specs=[_rows((NB, DIN)), _full((DIN, DH)), _full((1, DH)),
                  _full((DH, HC)), _full((1, HC)), _full((DH, HC)), _full((1, HC))],
        out_specs=[_rows((NB, HC)), _rows((NB, HC))],
        out_shape=[jax.ShapeDtypeStruct((NN, HC), jnp.float32)] * 2,
        interpret=_INTERP,
    )(x, win, bin_, wl, bl, wr, br)


def _alpha_body(q_r, ea_r, we_r, att_r, s_r):
    m = q_r[...] + _dot(ea_r[...], we_r[...])
    m = jnp.where(m >= 0.0, m, 0.2 * m)
    ma = m * att_r[...]
    parts = [jnp.sum(ma[:, h * DH:(h + 1) * DH], axis=1, keepdims=True)
             for h in range(NH)]
    s_r[...] = jnp.exp(jnp.concatenate(parts, axis=1))


def _alpha(q, ea, we, att_flat):
    return pl.pallas_call(
        _alpha_body,
        grid=(NE // EB,),
        in_specs=[_rows((EB, HC)), _rows((EB, DE)), _full((DE, HC)), _full((1, HC))],
        out_specs=_rows((EB, NH)),
        out_shape=jax.ShapeDtypeStruct((NE, NH), jnp.float32),
        interpret=_INTERP,
    )(q, ea, we, att_flat)


def _norm(acc, den256, gb, lng, lnb):
    o = acc / (den256 + 1e-16) + gb
    mu = jnp.mean(o, axis=1, keepdims=True)
    v = jnp.mean((o - mu) ** 2, axis=1, keepdims=True)
    o = (o - mu) / jnp.sqrt(v + 1e-5) * lng + lnb
    return jnp.where(o > 0.0, o, jnp.exp(o) - 1.0)


def _agg_norm(lo, hi, den16, gb, lng, lnb):
    acc = jnp.concatenate([lo, hi], axis=1)
    o = jnp.concatenate(
        [acc[:, h * DH:(h + 1) * DH] / (den16[:, h:h + 1] + 1e-16)
         for h in range(NH)], axis=1) + gb
    mu = jnp.mean(o, axis=1, keepdims=True)
    v = jnp.mean((o - mu) ** 2, axis=1, keepdims=True)
    o = (o - mu) / jnp.sqrt(v + 1e-5) * lng + lnb
    return jnp.where(o > 0.0, o, jnp.exp(o) - 1.0)


def _mid_body(lo_r, hi_r, den_r, gb_r, lng_r, lnb_r, wl_r, bl_r, wr_r, br_r,
              h0_r, xl_r, xr_r):
    h0 = _agg_norm(lo_r[...], hi_r[...], den_r[...], gb_r[...],
                   lng_r[...], lnb_r[...])
    h0_r[...] = h0
    xl_r[...] = _dot(h0, wl_r[...]) + bl_r[...]
    xr_r[...] = _dot(h0, wr_r[...]) + br_r[...]


def _mid(lo, hi, den, gb, lng, lnb, wl, bl, wr, br):
    return pl.pallas_call(
        _mid_body,
        grid=(NN // NB,),
        in_specs=[_rows((NB, HW)), _rows((NB, HW)), _rows((NB, 16)),
                  _full((1, HC)), _full((1, HC)), _full((1, HC)),
                  _full((HC, HC)), _full((1, HC)), _full((HC, HC)), _full((1, HC))],
        out_specs=[_rows((NB, HC))] * 3,
        out_shape=[jax.ShapeDtypeStruct((NN, HC), jnp.float32)] * 3,
        interpret=_INTERP,
    )(lo, hi, den, gb, lng, lnb, wl, bl, wr, br)


def _post1_body(lo_r, hi_r, den_r, gb_r, lng_r, lnb_r, res_r, h1_r):
    h1_r[...] = _agg_norm(lo_r[...], hi_r[...], den_r[...], gb_r[...],
                          lng_r[...], lnb_r[...]) + res_r[...]


def _post1(lo, hi, den, gb, lng, lnb, res):
    return pl.pallas_call(
        _post1_body,
        grid=(NN // NB,),
        in_specs=[_rows((NB, HW)), _rows((NB, HW)), _rows((NB, 16)),
                  _full((1, HC)), _full((1, HC)), _full((1, HC)), _rows((NB, HC))],
        out_specs=_rows((NB, HC)),
        out_shape=jax.ShapeDtypeStruct((NN, HC), jnp.float32),
        interpret=_INTERP,
    )(lo, hi, den, gb, lng, lnb, res)


def _c1_body(hp_r, h_r, hn_r, pm_r, nm_r, w0_r, w1_r, w2_r, cb_r, g_r, b_r, t1_r):
    y = (_dot(hp_r[...] * pm_r[...], w0_r[...]) + _dot(h_r[...], w1_r[...])
         + _dot(hn_r[...] * nm_r[...], w2_r[...]) + cb_r[...])
    t1_r[...] = jnp.maximum(y * _BN_K * g_r[...] + b_r[...], 0.0)


def _conv1(hp, h, hn, pm, nm, w0, w1, w2, cb, g, b):
    return pl.pallas_call(
        _c1_body,
        grid=(NN // NB,),
        in_specs=[_rows((NB, HC))] * 3 + [_rows((NB, 1))] * 2
                 + [_full((HC, HC))] * 3 + [_full((1, HC))] * 3,
        out_specs=_rows((NB, HC)),
        out_shape=jax.ShapeDtypeStruct((NN, HC), jnp.float32),
        interpret=_INTERP,
    )(hp, h, hn, pm, nm, w0, w1, w2, cb, g, b)


def _c2_body(tp_r, t_r, tn_r, pm_r, nm_r, sel_r, lw2_r, res_r,
             w0_r, w1_r, w2_r, cb_r, g_r, b_r, t2_r):
    y = (_dot(tp_r[...] * pm_r[...], w0_r[...]) + _dot(t_r[...], w1_r[...])
         + _dot(tn_r[...] * nm_r[...], w2_r[...])
         + _dot_hi(sel_r[...], lw2_r[...]) + cb_r[...])
    t2_r[...] = jnp.maximum(y * _BN_K * g_r[...] + b_r[...] + res_r[...], 0.0)


def _conv2(tp, t, tn, pm, nm, sel, lw2, res, w0, w1, w2, cb, g, b):
    return pl.pallas_call(
        _c2_body,
        grid=(NN // NB,),
        in_specs=[_rows((NB, HC))] * 3 + [_rows((NB, 1))] * 2
                 + [_rows((NB, NG)), _full((NG, HC)), _rows((NB, HC))]
                 + [_full((HC, HC))] * 3 + [_full((1, HC))] * 3,
        out_specs=_rows((NB, HC)),
        out_shape=jax.ShapeDtypeStruct((NN, HC), jnp.float32),
        interpret=_INTERP,
    )(tp, t, tn, pm, nm, sel, lw2, res, w0, w1, w2, cb, g, b)


def _leak_body(hl_r, t1l_r, cnt_r, tv_r, w0c1_r, c1b_r, g1_r, b1_r,
               w0c2_r, w1c2_r, w2c2_r, c2b_r, g2_r, b2_r,
               lw2_r, psum_r, pmax_r):
    cb = cnt_r[...]
    tv = tv_r[...]
    l1 = jnp.maximum((_dot(hl_r[...], w0c1_r[...]) + c1b_r[...]) * _BN_K
                     * g1_r[...] + b1_r[...], 0.0)
    m1 = ((cb > 0.0) & (cb < float(NN)) & (cb < tv)).astype(jnp.float32)
    l1m = l1 * m1
    p1 = jnp.maximum((_dot(t1l_r[...], w0c2_r[...]) + _dot(l1m, w1c2_r[...])
                      + c2b_r[...]) * _BN_K * g2_r[...] + b2_r[...], 0.0) * m1
    m2 = ((cb + 1.0 < tv) & (cb + 1.0 < float(NN))).astype(jnp.float32)
    p2 = jnp.maximum((_dot(l1m, w0c2_r[...]) + c2b_r[...]) * _BN_K
                     * g2_r[...] + b2_r[...], 0.0) * m2
    lw2_r[...] = _dot(l1m, w2c2_r[...])
    psum_r[...] = p1 + p2
    pmax_r[...] = jnp.maximum(p1, p2)


def _leak(hl, t1l, cnt, tv, w0c1, c1b, g1, b1, w0c2, w1c2, w2c2, c2b, g2, b2):
    return pl.pallas_call(
        _leak_body,
        in_specs=[_full((NG, HC))] * 2 + [_full((NG, 1)), _full((1, 1)),
                  _full((HC, HC)), _full((1, HC)), _full((1, HC)), _full((1, HC)),
                  _full((HC, HC)), _full((HC, HC)), _full((HC, HC)),
                  _full((1, HC)), _full((1, HC)), _full((1, HC))],
        out_specs=[_full((NG, HC))] * 3,
        out_shape=[jax.ShapeDtypeStruct((NG, HC), jnp.float32)] * 3,
        interpret=_INTERP,
    )(hl, t1l, cnt, tv, w0c1, c1b, g1, b1, w0c2, w1c2, w2c2, c2b, g2, b2)


def _pool_body(t2_r, oht_r, oh_r, psum_r, pmax_r, tv_r,
               w1_r, b1_r, w2_r, b2_r, w3_r, b3_r, out_r):
    t2 = t2_r[...]
    sums = _dot_hi(oht_r[...], t2)
    oh = oh_r[...]
    maxs = [jnp.max(t2 * oh[:, b:b + 1], axis=0, keepdims=True)
            for b in range(NG)]
    mx = jnp.concatenate(maxs, axis=0)
    mean = (sums + psum_r[...]) / tv_r[...]
    mx = jnp.maximum(mx, pmax_r[...])
    g = jnp.concatenate([mean, mx], axis=1)
    g = jnp.maximum(_dot(g, w1_r[...]) + b1_r[...], 0.0)
    g = jnp.maximum(_dot(g, w2_r[...]) + b2_r[...], 0.0)
    out_r[...] = _dot(g, w3_r[...]) + b3_r[...]


def _pool(t2, oht, oh, psum, pmax, tv, w1, b1, w2, b2, w3, b3):
    return pl.pallas_call(
        _pool_body,
        in_specs=[_full((NN, HC)), _full((NG, NN)), _full((NN, NG)),
                  _full((NG, HC)), _full((NG, HC)), _full((1, 1)),
                  _full((2 * HC, HC)), _full((1, HC)),
                  _full((HC, DH)), _full((1, DH)),
                  _full((DH, 1)), _full((1, 1))],
        out_specs=_full((NG, 1)),
        out_shape=jax.ShapeDtypeStruct((NG, 1), jnp.float32),
        interpret=_INTERP,
    )(t2, oht, oh, psum, pmax, tv, w1, b1, w2, b2, w3, b3)


_NC = 2
_NS = 16
CH = 80
HW = HC // 2
NNP = 10240
RPT = NNP // _NS


def _sc_mesh():
    return plsc.VectorSubcoreMesh(core_axis_name="c", subcore_axis_name="s")


QCH = 40


def _qadd_body(xl_hbm, xr_hbm, src_hbm, dst_hbm, q_hbm,
               sidx0, didx0, sidx1, didx1, xlb0, xrb0, wb0, xlb1, xrb1, wb1,
               gl0, gr0, gl1, gr1, w0, w1, is0, is1):
    c = lax.axis_index("c")
    s = lax.axis_index("s")
    wid = s * _NC + c
    nper = NE // (_NC * _NS)
    base0 = wid * nper
    nch = nper // QCH
    sidx = [sidx0, sidx1]
    didx = [didx0, didx1]
    xlb = [xlb0, xlb1]
    xrb = [xrb0, xrb1]
    wb = [wb0, wb1]
    gl = [gl0, gl1]
    gr = [gr0, gr1]
    ws = [w0, w1]
    iss = [is0, is1]

    def load_idx(k, p):
        base = base0 + k * QCH
        pltpu.async_copy(src_hbm.at[pl.ds(base, QCH)], sidx[p], iss[p])
        pltpu.async_copy(dst_hbm.at[pl.ds(base, QCH)], didx[p], iss[p])

    def wait_idx(p):
        pltpu.make_async_copy(src_hbm.at[pl.ds(0, QCH)], sidx[p], iss[p]).wait()
        pltpu.make_async_copy(dst_hbm.at[pl.ds(0, QCH)], didx[p], iss[p]).wait()

    def gather(p):
        pltpu.async_copy(xl_hbm.at[sidx[p]], xlb[p], gl[p])
        pltpu.async_copy(xr_hbm.at[didx[p]], xrb[p], gr[p])

    def wait_gather(p):
        pltpu.make_async_copy(xl_hbm.at[sidx[p]], xlb[p], gl[p]).wait()
        pltpu.make_async_copy(xr_hbm.at[didx[p]], xrb[p], gr[p]).wait()

    load_idx(0, 0)
    wait_idx(0)
    gather(0)
    load_idx(1, 1)
    wait_idx(1)
    gather(1)

    def body(m, carry):
        for p in range(2):
            k = 2 * m + p
            wait_gather(p)

            @pl.when(k + 2 < nch)
            def _():
                load_idx(k + 2, p)

            @pl.when(k >= 2)
            def _():
                pltpu.make_async_copy(wb[p], q_hbm.at[pl.ds(0, QCH)], ws[p]).wait()

            def edge(e, cc):
                for j in range(HC // 16):
                    wb[p][e, pl.ds(16 * j, 16)] = (xlb[p][e, pl.ds(16 * j, 16)]
                                                   + xrb[p][e, pl.ds(16 * j, 16)])
                return cc
            lax.fori_loop(0, QCH, edge, 0)
            pltpu.async_copy(wb[p], q_hbm.at[pl.ds(base0 + k * QCH, QCH)], ws[p])

            @pl.when(k + 2 < nch)
            def _():
                wait_idx(p)
                gather(p)
        return carry
    lax.fori_loop(0, nch // 2, body, 0)
    pltpu.make_async_copy(wb[0], q_hbm.at[pl.ds(0, QCH)], ws[0]).wait()
    pltpu.make_async_copy(wb[1], q_hbm.at[pl.ds(0, QCH)], ws[1]).wait()


def _qadd(xl, xr, src, dst):
    f = pl.kernel(
        _qadd_body, mesh=_sc_mesh(),
        out_type=jax.ShapeDtypeStruct((NE, HC), jnp.float32),
        scratch_types=[pltpu.VMEM((QCH,), jnp.int32), pltpu.VMEM((QCH,), jnp.int32),
                       pltpu.VMEM((QCH,), jnp.int32), pltpu.VMEM((QCH,), jnp.int32),
                       pltpu.VMEM((QCH, HC), jnp.float32),
                       pltpu.VMEM((QCH, HC), jnp.float32),
                       pltpu.VMEM((QCH, HC), jnp.float32),
                       pltpu.VMEM((QCH, HC), jnp.float32),
                       pltpu.VMEM((QCH, HC), jnp.float32),
                       pltpu.VMEM((QCH, HC), jnp.float32)]
                      + [pltpu.SemaphoreType.DMA] * 8,
    )
    return f(xl, xr, src, dst)


NDR = NNP // 8


SCH = 40


def _scat_body(xl2_hbm, src_hbm, dst_hbm, sflat_hbm, zin_hbm, lo_hbm, hi_hbm,
               dfl_hbm, acc, dtab,
               sidx0, didx0, idx20, dS0, d30, sbuf0, xlb0, cbuf0, cbuf20,
               sidx1, didx1, idx21, dS1, d31, sbuf1, xlb1, cbuf1, cbuf21,
               is0, is1, g0, g1, sa0, sa1, sb0, sb1):
    c = lax.axis_index("c")
    s = lax.axis_index("s")
    pltpu.sync_copy(zin_hbm, acc.at[pl.ds(s * RPT, RPT)])
    pltpu.sync_copy(zin_hbm.at[pl.ds(0, NDR // _NS)],
                    dtab.at[pl.ds(s * (NDR // _NS), NDR // _NS)])
    lane = jnp.arange(16, dtype=jnp.int32)
    zero16 = jnp.zeros((16,), jnp.float32)
    plsc.subcore_barrier()

    nper = NE // _NS
    base0 = s * nper
    nch = nper // SCH
    sidx = [sidx0, sidx1]
    didx = [didx0, didx1]
    idx2 = [idx20, idx21]
    dS = [dS0, dS1]
    d3 = [d30, d31]
    sbuf = [sbuf0, sbuf1]
    xlb = [xlb0, xlb1]
    cbuf = [cbuf0, cbuf1]
    cbuf2 = [cbuf20, cbuf21]
    iss = [is0, is1]
    gs = [g0, g1]
    sca = [sa0, sa1]
    scb = [sb0, sb1]
    _OFF = (0, 16, 24)

    def load_idx(k, p):
        base = base0 + k * SCH
        pltpu.async_copy(src_hbm.at[pl.ds(base, SCH)], sidx[p], iss[p])
        pltpu.async_copy(dst_hbm.at[pl.ds(base, SCH)], didx[p].at[pl.ds(0, SCH)], iss[p])
        pltpu.async_copy(sflat_hbm.at[pl.ds(NH * base, NH * SCH)],
                         sbuf[p].at[pl.ds(0, NH * SCH)], iss[p])

    def wait_idx(p):
        pltpu.make_async_copy(src_hbm.at[pl.ds(0, SCH)], sidx[p], iss[p]).wait()
        pltpu.make_async_copy(dst_hbm.at[pl.ds(0, SCH)],
                              didx[p].at[pl.ds(0, SCH)], iss[p]).wait()
        pltpu.make_async_copy(sflat_hbm.at[pl.ds(0, NH * SCH)],
                              sbuf[p].at[pl.ds(0, NH * SCH)], iss[p]).wait()

    def prep_gather(p):
        for v in _OFF:
            idx2[p][pl.ds(v, 16)] = sidx[p][pl.ds(v, 16)] * 2 + c
        pltpu.async_copy(xl2_hbm.at[idx2[p]], xlb[p], gs[p])

    def edges(p):
        for g in range(3):
            dvec = didx[p][pl.ds(16 * g, 16)]
            for i in range(16 if g < 2 else 8):
                e = 16 * g + i
                sv = sbuf[p][pl.ds(NH * e, 16)]
                sp0 = sv[jnp.broadcast_to(2 * c, (16,))]
                sp1 = sv[jnp.broadcast_to(2 * c + 1, (16,))]
                for j in range(HW // 16):
                    sp = sp0 if j < (HW // 32) else sp1
                    cbuf[p][e, pl.ds(16 * j, 16)] = sp * xlb[p][e, pl.ds(16 * j, 16)]
                for j in range(8):
                    cbuf2[p][e, pl.ds(16 * j, 16)] = zero16
                off = 16 * (dvec[i] & 7)
                cbuf2[p][e, pl.ds(off, 16)] = jnp.where(lane < NH, sv, 0.0)

    for p in range(2):
        load_idx(p, p)
        wait_idx(p)
        prep_gather(p)

    def body(m, carry):
        for p in range(2):
            k = 2 * m + p
            pltpu.make_async_copy(xl2_hbm.at[idx2[p]], xlb[p], gs[p]).wait()

            @pl.when(k >= 2)
            def _():
                pltpu.make_async_copy(cbuf[p], acc.at[dS[p]], sca[p]).wait()
                pltpu.make_async_copy(cbuf2[p], dtab.at[d3[p]], scb[p]).wait()

            edges(p)
            for v in _OFF:
                dv = didx[p][pl.ds(v, 16)]
                dS[p][pl.ds(v, 16)] = dv
                d3[p][pl.ds(v, 16)] = lax.shift_right_logical(dv, 3)
            pltpu.async_copy(cbuf[p], acc.at[dS[p]], sca[p], add=True)
            pltpu.async_copy(cbuf2[p], dtab.at[d3[p]], scb[p], add=True)

            @pl.when(k + 2 < nch)
            def _():
                load_idx(k + 2, p)
                wait_idx(p)
                prep_gather(p)
        return carry
    lax.fori_loop(0, nch // 2, body, 0)
    for p in range(2):
        pltpu.make_async_copy(cbuf[p], acc.at[dS[p]], sca[p]).wait()
        pltpu.make_async_copy(cbuf2[p], dtab.at[d3[p]], scb[p]).wait()
    plsc.subcore_barrier()

    @pl.when(c == 0)
    def _():
        pltpu.sync_copy(acc.at[pl.ds(s * RPT, RPT)], lo_hbm.at[pl.ds(s * RPT, RPT)])
        pltpu.sync_copy(dtab.at[pl.ds(s * (NDR // _NS), NDR // _NS)],
                        dfl_hbm.at[pl.ds(s * (NDR // _NS), NDR // _NS)])

    @pl.when(c == 1)
    def _():
        pltpu.sync_copy(acc.at[pl.ds(s * RPT, RPT)], hi_hbm.at[pl.ds(s * RPT, RPT)])


def _scatter(xl2, src, dst, s_flat, zin):
    per_parity = [pltpu.VMEM((SCH,), jnp.int32),
                  pltpu.VMEM((SCH + 16,), jnp.int32),
                  pltpu.VMEM((SCH,), jnp.int32),
                  pltpu.VMEM((SCH,), jnp.int32),
                  pltpu.VMEM((SCH,), jnp.int32),
                  pltpu.VMEM((NH * SCH + 16,), jnp.float32),
                  pltpu.VMEM((SCH, HW), jnp.float32),
                  pltpu.VMEM((SCH, HW), jnp.float32),
                  pltpu.VMEM((SCH, 128), jnp.float32)]
    f = pl.kernel(
        _scat_body, mesh=_sc_mesh(),
        out_type=[jax.ShapeDtypeStruct((NNP, HW), jnp.float32),
                  jax.ShapeDtypeStruct((NNP, HW), jnp.float32),
                  jax.ShapeDtypeStruct((NDR, 128), jnp.float32)],
        scratch_types=[pltpu.VMEM_SHARED((NNP, HW), jnp.float32),
                       pltpu.VMEM_SHARED((NDR, 128), jnp.float32)]
                      + per_parity + per_parity
                      + [pltpu.SemaphoreType.DMA] * 8,
    )
    return f(xl2, src, dst, s_flat, zin)


def _edge_aggregate(xl, xr, ea, src, dst, we, att_flat, zin):
    q = _qadd(xl, xr, src, dst)
    s = _alpha(q, ea, we, att_flat)
    lo, hi, dfl = _scatter(xl.reshape(2 * NN, HW), src, dst,
                           s.reshape(NE * NH), zin)
    den16 = dfl.reshape(NNP, 16)
    return lo[:NN], hi[:NN], den16[:NN]


def kernel(x, edge_index, edge_attr, batch, params):
    p = params
    src = edge_index[0]
    dst = edge_index[1]
    r1 = lambda a: a.reshape(1, -1)

    zin = jnp.zeros((RPT, HW), jnp.float32)

    xl0, xr0 = _pre(x, p['Win'], r1(p['bin']), p['g0_Wl'], r1(p['g0_bl']),
                    p['g0_Wr'], r1(p['g0_br']))
    lo0, hi0, den0 = _edge_aggregate(xl0, xr0, edge_attr, src, dst,
                                     p['g0_We'], r1(p['g0_att']), zin)
    h0, xl1, xr1 = _mid(lo0, hi0, den0, r1(p['g0_bias']), r1(p['ln0_g']),
                        r1(p['ln0_b']), p['g1_Wl'], r1(p['g1_bl']),
                        p['g1_Wr'], r1(p['g1_br']))
    lo1, hi1, den1 = _edge_aggregate(xl1, xr1, edge_attr, src, dst,
                                     p['g1_We'], r1(p['g1_att']), zin)
    h1 = _post1(lo1, hi1, den1, r1(p['g1_bias']), r1(p['ln1_g']),
                r1(p['ln1_b']), h0)

    idx = jnp.arange(NN, dtype=jnp.int32)
    o = jnp.searchsorted(batch, jnp.arange(NG + 1, dtype=jnp.int32)).astype(jnp.int32)
    cnt = (o[1:] - o[:-1]).astype(jnp.float32)
    tv = jnp.max(cnt)
    same_next = (batch[1:] == batch[:-1])
    pm = jnp.concatenate([jnp.zeros((1,), jnp.bool_), same_next]).astype(jnp.float32)[:, None]
    nm = jnp.concatenate([same_next, jnp.zeros((1,), jnp.bool_)]).astype(jnp.float32)[:, None]
    z1 = jnp.zeros((1, HC), jnp.float32)
    shift = lambda a: (jnp.concatenate([z1, a[:-1]], 0), jnp.concatenate([a[1:], z1], 0))

    lastrow = jnp.clip(o[1:] - 1, 0, NN - 1)
    sel = (idx[:, None] == (o[1:] - 1)[None, :]).astype(jnp.float32)
    oh = (batch[:, None] == jnp.arange(NG, dtype=jnp.int32)[None, :]).astype(jnp.float32)
    oht = oh.T

    wc1 = [p['c1_w'][:, :, k].T for k in range(3)]
    wc2 = [p['c2_w'][:, :, k].T for k in range(3)]

    h1p, h1n = shift(h1)
    t1 = _conv1(h1p, h1, h1n, pm, nm, wc1[0], wc1[1], wc1[2],
                r1(p['c1_b']), r1(p['bn1_g']), r1(p['bn1_b']))

    hl = h1[lastrow]
    t1l = t1[lastrow]
    lw2, psum, pmax = _leak(hl, t1l, cnt[:, None], tv.reshape(1, 1),
                            wc1[0], r1(p['c1_b']), r1(p['bn1_g']), r1(p['bn1_b']),
                            wc2[0], wc2[1], wc2[2],
                            r1(p['c2_b']), r1(p['bn2_g']), r1(p['bn2_b']))

    t1p, t1n = shift(t1)
    t2 = _conv2(t1p, t1, t1n, pm, nm, sel, lw2, h1,
                wc2[0], wc2[1], wc2[2],
                r1(p['c2_b']), r1(p['bn2_g']), r1(p['bn2_b']))

    out = _pool(t2, oht, oh, psum, pmax, tv.reshape(1, 1),
                p['w1'], r1(p['b1']), p['w2'], r1(p['b2']),
                p['w3'], r1(p['b3']))
    return out.reshape(NG)


# speedup vs baseline: 21.0624x; 1.0001x over previous
import jax
import jax.numpy as jnp
from jax import lax
from jax.experimental import pallas as pl
from jax.experimental.pallas import tpu as pltpu
from jax.experimental.pallas import tpu_sc as plsc

NN = 10000
NE = 320000
DIN = 128
DE = 4
NH = 4
DH = 64
HC = 256
NG = 8
_BN_K = 1.0 / (1.0 + 1e-5) ** 0.5

NB = 1000
EB = 4000


def _dot(a, b):
    return jnp.dot(a, b, preferred_element_type=jnp.float32)


def _dot_hi(a, b):
    return jnp.dot(a, b, preferred_element_type=jnp.float32,
                   precision=lax.Precision.HIGHEST)


def _full(shape):
    return pl.BlockSpec(shape, lambda *a: tuple(0 for _ in shape))


def _rows(shape):
    return pl.BlockSpec(shape, lambda i: (i,) + tuple(0 for _ in shape[1:]))


def _pre_body(x_r, win_r, bin_r, wl_r, bl_r, wr_r, br_r, xl_r, xr_r):
    h = _dot(x_r[...], win_r[...]) + bin_r[...]
    xl_r[...] = _dot(h, wl_r[...]) + bl_r[...]
    xr_r[...] = _dot(h, wr_r[...]) + br_r[...]


def _pre(x, win, bin_, wl, bl, wr, br):
    return pl.pallas_call(
        _pre_body,
        grid=(NN // NB,),
        in_specs=[_rows((NB, DIN)), _full((DIN, DH)), _full((1, DH)),
                  _full((DH, HC)), _full((1, HC)), _full((DH, HC)), _full((1, HC))],
        out_specs=[_rows((NB, HC)), _rows((NB, HC))],
        out_shape=[jax.ShapeDtypeStruct((NN, HC), jnp.float32)] * 2,
    )(x, win, bin_, wl, bl, wr, br)


def _alpha_body(q_r, ea_r, we_r, att_r, s_r):
    m = q_r[...] + _dot(ea_r[...], we_r[...])
    m = jnp.where(m >= 0.0, m, 0.2 * m)
    ma = m * att_r[...]
    parts = [jnp.sum(ma[:, h * DH:(h + 1) * DH], axis=1, keepdims=True)
             for h in range(NH)]
    s_r[...] = jnp.exp(jnp.concatenate(parts, axis=1))


def _alpha(q, ea, we, att_flat):
    return pl.pallas_call(
        _alpha_body,
        grid=(NE // EB,),
        in_specs=[_rows((EB, HC)), _rows((EB, DE)), _full((DE, HC)), _full((1, HC))],
        out_specs=_rows((EB, NH)),
        out_shape=jax.ShapeDtypeStruct((NE, NH), jnp.float32),
    )(q, ea, we, att_flat)


def _agg_norm(lo, hi, den16, gb, lng, lnb):
    acc = jnp.concatenate([lo, hi], axis=1)
    o = jnp.concatenate(
        [acc[:, h * DH:(h + 1) * DH] / (den16[:, h:h + 1] + 1e-16)
         for h in range(NH)], axis=1) + gb
    mu = jnp.mean(o, axis=1, keepdims=True)
    v = jnp.mean((o - mu) ** 2, axis=1, keepdims=True)
    o = (o - mu) / jnp.sqrt(v + 1e-5) * lng + lnb
    return jnp.where(o > 0.0, o, jnp.exp(o) - 1.0)


def _mid_body(lo_r, hi_r, den_r, gb_r, lng_r, lnb_r, wl_r, bl_r, wr_r, br_r,
              h0_r, xl_r, xr_r):
    h0 = _agg_norm(lo_r[...], hi_r[...], den_r[...], gb_r[...],
                   lng_r[...], lnb_r[...])
    h0_r[...] = h0
    xl_r[...] = _dot(h0, wl_r[...]) + bl_r[...]
    xr_r[...] = _dot(h0, wr_r[...]) + br_r[...]


def _mid(lo, hi, den, gb, lng, lnb, wl, bl, wr, br):
    return pl.pallas_call(
        _mid_body,
        grid=(NN // NB,),
        in_specs=[_rows((NB, HW)), _rows((NB, HW)), _rows((NB, 16)),
                  _full((1, HC)), _full((1, HC)), _full((1, HC)),
                  _full((HC, HC)), _full((1, HC)), _full((HC, HC)), _full((1, HC))],
        out_specs=[_rows((NB, HC))] * 3,
        out_shape=[jax.ShapeDtypeStruct((NN, HC), jnp.float32)] * 3,
    )(lo, hi, den, gb, lng, lnb, wl, bl, wr, br)


def _post1_body(lo_r, hi_r, den_r, gb_r, lng_r, lnb_r, res_r, h1_r):
    h1_r[...] = _agg_norm(lo_r[...], hi_r[...], den_r[...], gb_r[...],
                          lng_r[...], lnb_r[...]) + res_r[...]


def _post1(lo, hi, den, gb, lng, lnb, res):
    return pl.pallas_call(
        _post1_body,
        grid=(NN // NB,),
        in_specs=[_rows((NB, HW)), _rows((NB, HW)), _rows((NB, 16)),
                  _full((1, HC)), _full((1, HC)), _full((1, HC)), _rows((NB, HC))],
        out_specs=_rows((NB, HC)),
        out_shape=jax.ShapeDtypeStruct((NN, HC), jnp.float32),
    )(lo, hi, den, gb, lng, lnb, res)


def _c1_body(hp_r, h_r, hn_r, pm_r, nm_r, w0_r, w1_r, w2_r, cb_r, g_r, b_r, t1_r):
    y = (_dot(hp_r[...] * pm_r[...], w0_r[...]) + _dot(h_r[...], w1_r[...])
         + _dot(hn_r[...] * nm_r[...], w2_r[...]) + cb_r[...])
    t1_r[...] = jnp.maximum(y * _BN_K * g_r[...] + b_r[...], 0.0)


def _conv1(hp, h, hn, pm, nm, w0, w1, w2, cb, g, b):
    return pl.pallas_call(
        _c1_body,
        grid=(NN // NB,),
        in_specs=[_rows((NB, HC))] * 3 + [_rows((NB, 1))] * 2
                 + [_full((HC, HC))] * 3 + [_full((1, HC))] * 3,
        out_specs=_rows((NB, HC)),
        out_shape=jax.ShapeDtypeStruct((NN, HC), jnp.float32),
    )(hp, h, hn, pm, nm, w0, w1, w2, cb, g, b)


def _c2_body(tp_r, t_r, tn_r, pm_r, nm_r, sel_r, lw2_r, res_r,
             w0_r, w1_r, w2_r, cb_r, g_r, b_r, t2_r):
    y = (_dot(tp_r[...] * pm_r[...], w0_r[...]) + _dot(t_r[...], w1_r[...])
         + _dot(tn_r[...] * nm_r[...], w2_r[...])
         + _dot_hi(sel_r[...], lw2_r[...]) + cb_r[...])
    t2_r[...] = jnp.maximum(y * _BN_K * g_r[...] + b_r[...] + res_r[...], 0.0)


def _conv2(tp, t, tn, pm, nm, sel, lw2, res, w0, w1, w2, cb, g, b):
    return pl.pallas_call(
        _c2_body,
        grid=(NN // NB,),
        in_specs=[_rows((NB, HC))] * 3 + [_rows((NB, 1))] * 2
                 + [_rows((NB, NG)), _full((NG, HC)), _rows((NB, HC))]
                 + [_full((HC, HC))] * 3 + [_full((1, HC))] * 3,
        out_specs=_rows((NB, HC)),
        out_shape=jax.ShapeDtypeStruct((NN, HC), jnp.float32),
    )(tp, t, tn, pm, nm, sel, lw2, res, w0, w1, w2, cb, g, b)


def _leak_body(hl_r, t1l_r, cnt_r, tv_r, w0c1_r, c1b_r, g1_r, b1_r,
               w0c2_r, w1c2_r, w2c2_r, c2b_r, g2_r, b2_r,
               lw2_r, psum_r, pmax_r):
    cb = cnt_r[...]
    tv = tv_r[...]
    l1 = jnp.maximum((_dot(hl_r[...], w0c1_r[...]) + c1b_r[...]) * _BN_K
                     * g1_r[...] + b1_r[...], 0.0)
    m1 = ((cb > 0.0) & (cb < float(NN)) & (cb < tv)).astype(jnp.float32)
    l1m = l1 * m1
    p1 = jnp.maximum((_dot(t1l_r[...], w0c2_r[...]) + _dot(l1m, w1c2_r[...])
                      + c2b_r[...]) * _BN_K * g2_r[...] + b2_r[...], 0.0) * m1
    m2 = ((cb + 1.0 < tv) & (cb + 1.0 < float(NN))).astype(jnp.float32)
    p2 = jnp.maximum((_dot(l1m, w0c2_r[...]) + c2b_r[...]) * _BN_K
                     * g2_r[...] + b2_r[...], 0.0) * m2
    lw2_r[...] = _dot(l1m, w2c2_r[...])
    psum_r[...] = p1 + p2
    pmax_r[...] = jnp.maximum(p1, p2)


def _leak(hl, t1l, cnt, tv, w0c1, c1b, g1, b1, w0c2, w1c2, w2c2, c2b, g2, b2):
    return pl.pallas_call(
        _leak_body,
        in_specs=[_full((NG, HC))] * 2 + [_full((NG, 1)), _full((1, 1)),
                  _full((HC, HC)), _full((1, HC)), _full((1, HC)), _full((1, HC)),
                  _full((HC, HC)), _full((HC, HC)), _full((HC, HC)),
                  _full((1, HC)), _full((1, HC)), _full((1, HC))],
        out_specs=[_full((NG, HC))] * 3,
        out_shape=[jax.ShapeDtypeStruct((NG, HC), jnp.float32)] * 3,
    )(hl, t1l, cnt, tv, w0c1, c1b, g1, b1, w0c2, w1c2, w2c2, c2b, g2, b2)


def _pool_body(t2_r, oht_r, oh_r, psum_r, pmax_r, tv_r,
               w1_r, b1_r, w2_r, b2_r, w3_r, b3_r, out_r):
    t2 = t2_r[...]
    sums = _dot_hi(oht_r[...], t2)
    oh = oh_r[...]
    maxs = [jnp.max(t2 * oh[:, b:b + 1], axis=0, keepdims=True)
            for b in range(NG)]
    mx = jnp.concatenate(maxs, axis=0)
    mean = (sums + psum_r[...]) / tv_r[...]
    mx = jnp.maximum(mx, pmax_r[...])
    g = jnp.concatenate([mean, mx], axis=1)
    g = jnp.maximum(_dot(g, w1_r[...]) + b1_r[...], 0.0)
    g = jnp.maximum(_dot(g, w2_r[...]) + b2_r[...], 0.0)
    out_r[...] = _dot(g, w3_r[...]) + b3_r[...]


def _pool(t2, oht, oh, psum, pmax, tv, w1, b1, w2, b2, w3, b3):
    return pl.pallas_call(
        _pool_body,
        in_specs=[_full((NN, HC)), _full((NG, NN)), _full((NN, NG)),
                  _full((NG, HC)), _full((NG, HC)), _full((1, 1)),
                  _full((2 * HC, HC)), _full((1, HC)),
                  _full((HC, DH)), _full((1, DH)),
                  _full((DH, 1)), _full((1, 1))],
        out_specs=_full((NG, 1)),
        out_shape=jax.ShapeDtypeStruct((NG, 1), jnp.float32),
    )(t2, oht, oh, psum, pmax, tv, w1, b1, w2, b2, w3, b3)


_NC = 2
_NS = 16
HW = HC // 2
NNP = 10240
RPT = NNP // _NS


def _sc_mesh():
    return plsc.VectorSubcoreMesh(core_axis_name="c", subcore_axis_name="s")


QCH = 40


def _qadd_body(xl_hbm, xr_hbm, src_hbm, dst_hbm, q_hbm,
               sidx0, didx0, sidx1, didx1, xlb0, xrb0, wb0, xlb1, xrb1, wb1,
               gl0, gr0, gl1, gr1, w0, w1, is0, is1):
    c = lax.axis_index("c")
    s = lax.axis_index("s")
    wid = s * _NC + c
    nper = NE // (_NC * _NS)
    base0 = wid * nper
    nch = nper // QCH
    sidx = [sidx0, sidx1]
    didx = [didx0, didx1]
    xlb = [xlb0, xlb1]
    xrb = [xrb0, xrb1]
    wb = [wb0, wb1]
    gl = [gl0, gl1]
    gr = [gr0, gr1]
    ws = [w0, w1]
    iss = [is0, is1]

    def load_idx(k, p):
        base = base0 + k * QCH
        pltpu.async_copy(src_hbm.at[pl.ds(base, QCH)], sidx[p], iss[p])
        pltpu.async_copy(dst_hbm.at[pl.ds(base, QCH)], didx[p], iss[p])

    def wait_idx(p):
        pltpu.make_async_copy(src_hbm.at[pl.ds(0, QCH)], sidx[p], iss[p]).wait()
        pltpu.make_async_copy(dst_hbm.at[pl.ds(0, QCH)], didx[p], iss[p]).wait()

    def gather(p):
        pltpu.async_copy(xl_hbm.at[sidx[p]], xlb[p], gl[p])
        pltpu.async_copy(xr_hbm.at[didx[p]], xrb[p], gr[p])

    def wait_gather(p):
        pltpu.make_async_copy(xl_hbm.at[sidx[p]], xlb[p], gl[p]).wait()
        pltpu.make_async_copy(xr_hbm.at[didx[p]], xrb[p], gr[p]).wait()

    load_idx(0, 0)
    wait_idx(0)
    gather(0)
    load_idx(1, 1)
    wait_idx(1)
    gather(1)

    def body(m, carry):
        for p in range(2):
            k = 2 * m + p
            wait_gather(p)

            @pl.when(k + 2 < nch)
            def _():
                load_idx(k + 2, p)

            @pl.when(k >= 2)
            def _():
                pltpu.make_async_copy(wb[p], q_hbm.at[pl.ds(0, QCH)], ws[p]).wait()

            def edge(e, cc):
                for j in range(HC // 16):
                    wb[p][e, pl.ds(16 * j, 16)] = (xlb[p][e, pl.ds(16 * j, 16)]
                                                   + xrb[p][e, pl.ds(16 * j, 16)])
                return cc
            lax.fori_loop(0, QCH, edge, 0)
            pltpu.async_copy(wb[p], q_hbm.at[pl.ds(base0 + k * QCH, QCH)], ws[p])

            @pl.when(k + 2 < nch)
            def _():
                wait_idx(p)
                gather(p)
        return carry
    lax.fori_loop(0, nch // 2, body, 0)
    pltpu.make_async_copy(wb[0], q_hbm.at[pl.ds(0, QCH)], ws[0]).wait()
    pltpu.make_async_copy(wb[1], q_hbm.at[pl.ds(0, QCH)], ws[1]).wait()


def _qadd(xl, xr, src, dst):
    f = pl.kernel(
        _qadd_body, mesh=_sc_mesh(),
        out_type=jax.ShapeDtypeStruct((NE, HC), jnp.float32),
        scratch_types=[pltpu.VMEM((QCH,), jnp.int32), pltpu.VMEM((QCH,), jnp.int32),
                       pltpu.VMEM((QCH,), jnp.int32), pltpu.VMEM((QCH,), jnp.int32),
                       pltpu.VMEM((QCH, HC), jnp.float32),
                       pltpu.VMEM((QCH, HC), jnp.float32),
                       pltpu.VMEM((QCH, HC), jnp.float32),
                       pltpu.VMEM((QCH, HC), jnp.float32),
                       pltpu.VMEM((QCH, HC), jnp.float32),
                       pltpu.VMEM((QCH, HC), jnp.float32)]
                      + [pltpu.SemaphoreType.DMA] * 8,
    )
    return f(xl, xr, src, dst)


NDR = NNP // 8


SCH = 40


def _scat_body(xl2_hbm, src_hbm, dst_hbm, sflat_hbm, zin_hbm, lo_hbm, hi_hbm,
               dfl_hbm, acc, dtab,
               sidx0, didx0, idx20, dS0, d30, sbuf0, xlb0, cbuf0, cbuf20,
               sidx1, didx1, idx21, dS1, d31, sbuf1, xlb1, cbuf1, cbuf21,
               is0, is1, g0, g1, sa0, sa1, sb0, sb1):
    c = lax.axis_index("c")
    s = lax.axis_index("s")
    pltpu.sync_copy(zin_hbm, acc.at[pl.ds(s * RPT, RPT)])
    pltpu.sync_copy(zin_hbm.at[pl.ds(0, NDR // _NS)],
                    dtab.at[pl.ds(s * (NDR // _NS), NDR // _NS)])
    lane = jnp.arange(16, dtype=jnp.int32)
    zero16 = jnp.zeros((16,), jnp.float32)
    plsc.subcore_barrier()

    nper = NE // _NS
    base0 = s * nper
    nch = nper // SCH
    sidx = [sidx0, sidx1]
    didx = [didx0, didx1]
    idx2 = [idx20, idx21]
    dS = [dS0, dS1]
    d3 = [d30, d31]
    sbuf = [sbuf0, sbuf1]
    xlb = [xlb0, xlb1]
    cbuf = [cbuf0, cbuf1]
    cbuf2 = [cbuf20, cbuf21]
    iss = [is0, is1]
    gs = [g0, g1]
    sca = [sa0, sa1]
    scb = [sb0, sb1]
    _OFF = (0, 16, 24)

    def load_idx(k, p):
        base = base0 + k * SCH
        pltpu.async_copy(src_hbm.at[pl.ds(base, SCH)], sidx[p], iss[p])
        pltpu.async_copy(dst_hbm.at[pl.ds(base, SCH)], didx[p].at[pl.ds(0, SCH)], iss[p])
        pltpu.async_copy(sflat_hbm.at[pl.ds(NH * base, NH * SCH)],
                         sbuf[p].at[pl.ds(0, NH * SCH)], iss[p])

    def wait_idx(p):
        pltpu.make_async_copy(src_hbm.at[pl.ds(0, SCH)], sidx[p], iss[p]).wait()
        pltpu.make_async_copy(dst_hbm.at[pl.ds(0, SCH)],
                              didx[p].at[pl.ds(0, SCH)], iss[p]).wait()
        pltpu.make_async_copy(sflat_hbm.at[pl.ds(0, NH * SCH)],
                              sbuf[p].at[pl.ds(0, NH * SCH)], iss[p]).wait()

    def prep_gather(p):
        for v in _OFF:
            idx2[p][pl.ds(v, 16)] = sidx[p][pl.ds(v, 16)] * 2 + c
        pltpu.async_copy(xl2_hbm.at[idx2[p]], xlb[p], gs[p])

    def edges(p):
        for g in range(3):
            dvec = didx[p][pl.ds(16 * g, 16)]
            for i in range(16 if g < 2 else 8):
                e = 16 * g + i
                sv = sbuf[p][pl.ds(NH * e, 16)]
                sp0 = sv[jnp.broadcast_to(2 * c, (16,))]
                sp1 = sv[jnp.broadcast_to(2 * c + 1, (16,))]
                for j in range(HW // 16):
                    sp = sp0 if j < (HW // 32) else sp1
                    cbuf[p][e, pl.ds(16 * j, 16)] = sp * xlb[p][e, pl.ds(16 * j, 16)]
                for j in range(8):
                    cbuf2[p][e, pl.ds(16 * j, 16)] = zero16
                off = 16 * (dvec[i] & 7)
                cbuf2[p][e, pl.ds(off, 16)] = jnp.where(lane < NH, sv, 0.0)

    for p in range(2):
        load_idx(p, p)
        wait_idx(p)
        prep_gather(p)

    def body(m, carry):
        for p in range(2):
            k = 2 * m + p
            pltpu.make_async_copy(xl2_hbm.at[idx2[p]], xlb[p], gs[p]).wait()

            @pl.when(k >= 2)
            def _():
                pltpu.make_async_copy(cbuf[p], acc.at[dS[p]], sca[p]).wait()
                pltpu.make_async_copy(cbuf2[p], dtab.at[d3[p]], scb[p]).wait()

            edges(p)
            for v in _OFF:
                dv = didx[p][pl.ds(v, 16)]
                dS[p][pl.ds(v, 16)] = dv
                d3[p][pl.ds(v, 16)] = lax.shift_right_logical(dv, 3)
            pltpu.async_copy(cbuf[p], acc.at[dS[p]], sca[p], add=True)
            pltpu.async_copy(cbuf2[p], dtab.at[d3[p]], scb[p], add=True)

            @pl.when(k + 2 < nch)
            def _():
                load_idx(k + 2, p)
                wait_idx(p)
                prep_gather(p)
        return carry
    lax.fori_loop(0, nch // 2, body, 0)
    for p in range(2):
        pltpu.make_async_copy(cbuf[p], acc.at[dS[p]], sca[p]).wait()
        pltpu.make_async_copy(cbuf2[p], dtab.at[d3[p]], scb[p]).wait()
    plsc.subcore_barrier()

    @pl.when(c == 0)
    def _():
        pltpu.sync_copy(acc.at[pl.ds(s * RPT, RPT)], lo_hbm.at[pl.ds(s * RPT, RPT)])
        pltpu.sync_copy(dtab.at[pl.ds(s * (NDR // _NS), NDR // _NS)],
                        dfl_hbm.at[pl.ds(s * (NDR // _NS), NDR // _NS)])

    @pl.when(c == 1)
    def _():
        pltpu.sync_copy(acc.at[pl.ds(s * RPT, RPT)], hi_hbm.at[pl.ds(s * RPT, RPT)])


def _scatter(xl2, src, dst, s_flat, zin):
    per_parity = [pltpu.VMEM((SCH,), jnp.int32),
                  pltpu.VMEM((SCH + 16,), jnp.int32),
                  pltpu.VMEM((SCH,), jnp.int32),
                  pltpu.VMEM((SCH,), jnp.int32),
                  pltpu.VMEM((SCH,), jnp.int32),
                  pltpu.VMEM((NH * SCH + 16,), jnp.float32),
                  pltpu.VMEM((SCH, HW), jnp.float32),
                  pltpu.VMEM((SCH, HW), jnp.float32),
                  pltpu.VMEM((SCH, 128), jnp.float32)]
    f = pl.kernel(
        _scat_body, mesh=_sc_mesh(),
        out_type=[jax.ShapeDtypeStruct((NNP, HW), jnp.float32),
                  jax.ShapeDtypeStruct((NNP, HW), jnp.float32),
                  jax.ShapeDtypeStruct((NDR, 128), jnp.float32)],
        scratch_types=[pltpu.VMEM_SHARED((NNP, HW), jnp.float32),
                       pltpu.VMEM_SHARED((NDR, 128), jnp.float32)]
                      + per_parity + per_parity
                      + [pltpu.SemaphoreType.DMA] * 8,
    )
    return f(xl2, src, dst, s_flat, zin)


def _edge_aggregate(xl, xr, ea, src, dst, we, att_flat, zin):
    q = _qadd(xl, xr, src, dst)
    s = _alpha(q, ea, we, att_flat)
    lo, hi, dfl = _scatter(xl.reshape(2 * NN, HW), src, dst,
                           s.reshape(NE * NH), zin)
    den16 = dfl.reshape(NNP, 16)
    return lo[:NN], hi[:NN], den16[:NN]


def kernel(x, edge_index, edge_attr, batch, params):
    p = params
    src = edge_index[0]
    dst = edge_index[1]
    r1 = lambda a: a.reshape(1, -1)

    zin = jnp.zeros((RPT, HW), jnp.float32)

    xl0, xr0 = _pre(x, p['Win'], r1(p['bin']), p['g0_Wl'], r1(p['g0_bl']),
                    p['g0_Wr'], r1(p['g0_br']))
    lo0, hi0, den0 = _edge_aggregate(xl0, xr0, edge_attr, src, dst,
                                     p['g0_We'], r1(p['g0_att']), zin)
    h0, xl1, xr1 = _mid(lo0, hi0, den0, r1(p['g0_bias']), r1(p['ln0_g']),
                        r1(p['ln0_b']), p['g1_Wl'], r1(p['g1_bl']),
                        p['g1_Wr'], r1(p['g1_br']))
    lo1, hi1, den1 = _edge_aggregate(xl1, xr1, edge_attr, src, dst,
                                     p['g1_We'], r1(p['g1_att']), zin)
    h1 = _post1(lo1, hi1, den1, r1(p['g1_bias']), r1(p['ln1_g']),
                r1(p['ln1_b']), h0)

    idx = jnp.arange(NN, dtype=jnp.int32)
    o = jnp.searchsorted(batch, jnp.arange(NG + 1, dtype=jnp.int32)).astype(jnp.int32)
    cnt = (o[1:] - o[:-1]).astype(jnp.float32)
    tv = jnp.max(cnt)
    same_next = (batch[1:] == batch[:-1])
    pm = jnp.concatenate([jnp.zeros((1,), jnp.bool_), same_next]).astype(jnp.float32)[:, None]
    nm = jnp.concatenate([same_next, jnp.zeros((1,), jnp.bool_)]).astype(jnp.float32)[:, None]
    z1 = jnp.zeros((1, HC), jnp.float32)
    shift = lambda a: (jnp.concatenate([z1, a[:-1]], 0), jnp.concatenate([a[1:], z1], 0))

    lastrow = jnp.clip(o[1:] - 1, 0, NN - 1)
    sel = (idx[:, None] == (o[1:] - 1)[None, :]).astype(jnp.float32)
    oh = (batch[:, None] == jnp.arange(NG, dtype=jnp.int32)[None, :]).astype(jnp.float32)
    oht = oh.T

    wc1 = [p['c1_w'][:, :, k].T for k in range(3)]
    wc2 = [p['c2_w'][:, :, k].T for k in range(3)]

    h1p, h1n = shift(h1)
    t1 = _conv1(h1p, h1, h1n, pm, nm, wc1[0], wc1[1], wc1[2],
                r1(p['c1_b']), r1(p['bn1_g']), r1(p['bn1_b']))

    hl = h1[lastrow]
    t1l = t1[lastrow]
    lw2, psum, pmax = _leak(hl, t1l, cnt[:, None], tv.reshape(1, 1),
                            wc1[0], r1(p['c1_b']), r1(p['bn1_g']), r1(p['bn1_b']),
                            wc2[0], wc2[1], wc2[2],
                            r1(p['c2_b']), r1(p['bn2_g']), r1(p['bn2_b']))

    t1p, t1n = shift(t1)
    t2 = _conv2(t1p, t1, t1n, pm, nm, sel, lw2, h1,
                wc2[0], wc2[1], wc2[2],
                r1(p['c2_b']), r1(p['bn2_g']), r1(p['bn2_b']))

    out = _pool(t2, oht, oh, psum, pmax, tv.reshape(1, 1),
                p['w1'], r1(p['b1']), p['w2'], r1(p['b2']),
                p['w3'], r1(p['b3']))
    return out.reshape(NG)
